# baseline (speedup 1.0000x reference)
import functools

import jax
import jax.numpy as jnp
from jax.experimental import pallas as pl
from jax.experimental.pallas import tpu as pltpu

_BIG = 1e10


def _cparams(n):
    return pltpu.CompilerParams(dimension_semantics=("parallel",) * n)


def _iota2(shape, dim):
    return jax.lax.broadcasted_iota(jnp.int32, shape, dim)


def _fps(xyz, npoint):
    B0, N, _ = xyz.shape
    C = 128 if N >= 128 else N
    R = N // C
    planes = xyz.transpose(0, 2, 1).reshape(B0, 3, R, C)
    G = 2 if B0 % 2 == 0 else 1
    Bc = B0 // G

    def kern(p_ref, out_ref):
        X = p_ref[:, 0, :, :]
        Y = p_ref[:, 1, :, :]
        Z = p_ref[:, 2, :, :]
        flat = (_iota2((Bc, R, C), 1) * C + _iota2((Bc, R, C), 2))
        lane = _iota2((Bc, 1, npoint), 2)

        def red(x, op):
            return op(op(x, axis=2, keepdims=True), axis=1, keepdims=True)

        def step(t, carry):
            dists, far, CX, CY, CZ = carry
            sel = flat == far
            cx = red(jnp.where(sel, X, 0.0), jnp.sum)
            cy = red(jnp.where(sel, Y, 0.0), jnp.sum)
            cz = red(jnp.where(sel, Z, 0.0), jnp.sum)
            CX = jnp.where(lane == t, cx, CX)
            CY = jnp.where(lane == t, cy, CY)
            CZ = jnp.where(lane == t, cz, CZ)
            dx = X - cx
            dy = Y - cy
            dz = Z - cz
            d = dx * dx + dy * dy + dz * dz
            dists = jnp.minimum(dists, d)
            m = red(dists, jnp.max)
            far = red(jnp.where(dists == m, flat, N), jnp.min)
            return dists, far, CX, CY, CZ

        init = (
            jnp.full((Bc, R, C), _BIG, jnp.float32),
            jnp.zeros((Bc, 1, 1), jnp.int32),
            jnp.zeros((Bc, 1, npoint), jnp.float32),
            jnp.zeros((Bc, 1, npoint), jnp.float32),
            jnp.zeros((Bc, 1, npoint), jnp.float32),
        )
        _, _, CX, CY, CZ = jax.lax.fori_loop(0, npoint, step, init)
        out_ref[:, 0:1, :] = CX
        out_ref[:, 1:2, :] = CY
        out_ref[:, 2:3, :] = CZ

    return pl.pallas_call(
        kern,
        grid=(G,),
        in_specs=[pl.BlockSpec((Bc, 3, R, C), lambda i: (i, 0, 0, 0))],
        out_specs=pl.BlockSpec((Bc, 3, npoint), lambda i: (i, 0, 0)),
        out_shape=jax.ShapeDtypeStruct((B0, 3, npoint), jnp.float32),
        compiler_params=_cparams(1),
    )(planes)


def _group(q_xyz, cand_xyz_t, table, ws, ns, r2, qb, self_feat=None):
    Bc, nq, _ = q_xyz.shape
    nc, Ft = table.shape[1], table.shape[2]
    nblk = nq // qb
    S = 0 if self_feat is None else self_feat.shape[2]
    gw = 3 + S + (Ft - 3)
    outF = ws[-1][0].shape[1] if ws else gw

    wargs = []
    wspecs = []
    for (W, b) in ws:
        wargs += [W, b.reshape(1, -1)]
        wspecs += [
            pl.BlockSpec(W.shape, lambda i, j: (0, 0)),
            pl.BlockSpec((1, b.shape[0]), lambda i, j: (0, 0)),
        ]
    sargs = [] if self_feat is None else [self_feat]
    sspecs = [] if self_feat is None else [
        pl.BlockSpec((1, qb, S), lambda i, j: (i, j, 0))
    ]

    def kern(q_ref, cxt_ref, tab_ref, *rest):
        out_ref = rest[-1]
        rest = rest[:-1]
        self_blk = None
        if self_feat is not None:
            self_blk = rest[0][0]
            rest = rest[1:]
        qx = q_ref[0]
        dx = qx[:, 0:1] - cxt_ref[0, 0:1, :]
        dy = qx[:, 1:2] - cxt_ref[0, 1:2, :]
        dz = qx[:, 2:3] - cxt_ref[0, 2:3, :]
        d = dx * dx + dy * dy + dz * dz
        lane = _iota2((qb, nc), 1)

        ohs = []
        if r2 is not None:
            mask = d <= r2
            mf = mask.astype(jnp.float32)
            rank = mf
            s = 1
            while s < nc:
                rank = rank + jnp.where(lane >= s,
                                        pltpu.roll(rank, s, 1), 0.0)
                s *= 2
            rank = rank - mf
            count = jnp.sum(mf, axis=1, keepdims=True)
            oh0 = jnp.where(mask & (rank == 0.0), 1.0, 0.0)
            pad = jnp.where(count > 0.0, oh0,
                            (lane == 0).astype(jnp.float32))
            for k in range(ns):
                ohk = jnp.where(mask & (rank == float(k)), 1.0, 0.0)
                ohs.append(jnp.where(count > float(k), ohk, pad))
        else:
            for k in range(ns):
                mval = jnp.min(d, axis=1, keepdims=True)
                cur = jnp.min(jnp.where(d == mval, lane, nc), axis=1,
                              keepdims=True)
                d = jnp.where(lane == cur, _BIG, d)
                ohs.append((lane == cur).astype(jnp.float32))

        tab = tab_ref[0]
        if nc <= 512:
            oh = jnp.concatenate(ohs, axis=0)
            rows = jnp.dot(oh, tab, preferred_element_type=jnp.float32)
        else:
            rows = jnp.concatenate(
                [jnp.dot(o, tab, preferred_element_type=jnp.float32)
                 for o in ohs], axis=0)

        qxt = jnp.concatenate([qx] * ns, axis=0)
        parts = [rows[:, 0:3] - qxt]
        if self_blk is not None:
            parts.append(jnp.concatenate([self_blk] * ns, axis=0))
        parts.append(rows[:, 3:])
        h = jnp.concatenate(parts, axis=1)
        for (W, b) in zip(rest[0::2], rest[1::2]):
            h = jnp.maximum(
                jnp.dot(h, W[...], preferred_element_type=jnp.float32)
                + b[...], 0.0)
        h = h.reshape(ns, qb, outF)
        out_ref[0] = jnp.max(h, axis=0)

    return pl.pallas_call(
        kern,
        grid=(Bc, nblk),
        in_specs=[
            pl.BlockSpec((1, qb, 3), lambda i, j: (i, j, 0)),
            pl.BlockSpec((1, 3, nc), lambda i, j: (i, 0, 0)),
            pl.BlockSpec((1, nc, Ft), lambda i, j: (i, 0, 0)),
        ] + sspecs + wspecs,
        out_specs=pl.BlockSpec((1, qb, outF), lambda i, j: (i, j, 0)),
        out_shape=jax.ShapeDtypeStruct((Bc, nq, outF), jnp.float32),
        compiler_params=_cparams(2),
    )(q_xyz, cand_xyz_t, table, *sargs, *wargs)


def _mlp(x, ws, relu_last=True):
    Bc, rows, _ = x.shape
    outF = ws[-1][0].shape[1]
    wargs = []
    wspecs = []
    for (W, b) in ws:
        wargs += [W, b.reshape(1, -1)]
        wspecs += [
            pl.BlockSpec(W.shape, lambda i: (0, 0)),
            pl.BlockSpec((1, b.shape[0]), lambda i: (0, 0)),
        ]

    def kern(x_ref, *rest):
        out_ref = rest[-1]
        h = x_ref[0]
        wl = list(zip(rest[0:-1:2], rest[1:-1:2]))
        for li, (W, b) in enumerate(wl):
            h = jnp.dot(h, W[...], preferred_element_type=jnp.float32) + b[...]
            if relu_last or li < len(wl) - 1:
                h = jnp.maximum(h, 0.0)
        out_ref[0] = h

    return pl.pallas_call(
        kern,
        grid=(Bc,),
        in_specs=[pl.BlockSpec((1, rows, x.shape[2]), lambda i: (i, 0, 0))]
        + wspecs,
        out_specs=pl.BlockSpec((1, rows, outF), lambda i: (i, 0, 0)),
        out_shape=jax.ShapeDtypeStruct((Bc, rows, outF), jnp.float32),
        compiler_params=_cparams(1),
    )(x, *wargs)


def _fp_cls(fine_xyz, coarse_xyz_t, coarse_feat, fine_feat, fp_ws, cls_ws):
    Bc, N, _ = fine_xyz.shape
    M, F = coarse_feat.shape[1], coarse_feat.shape[2]
    S = fine_feat.shape[2]
    qb = 512
    nblk = N // qb
    layers = list(fp_ws) + list(cls_ws)
    n_relu = len(layers) - 1
    wargs = []
    wspecs = []
    for (W, b) in layers:
        wargs += [W, b.reshape(1, -1)]
        wspecs += [
            pl.BlockSpec(W.shape, lambda i, j: (0, 0)),
            pl.BlockSpec((1, b.shape[0]), lambda i, j: (0, 0)),
        ]

    def kern(fx_ref, cxt_ref, cf_ref, ff_ref, *rest):
        out_ref = rest[-1]
        wl = list(zip(rest[0:-1:2], rest[1:-1:2]))
        qx = fx_ref[0]
        dx = qx[:, 0:1] - cxt_ref[0, 0:1, :]
        dy = qx[:, 1:2] - cxt_ref[0, 1:2, :]
        dz = qx[:, 2:3] - cxt_ref[0, 2:3, :]
        d = dx * dx + dy * dy + dz * dz
        lane = _iota2((qb, M), 1)
        Wm = jnp.zeros((qb, M), jnp.float32)
        wsum = jnp.zeros((qb, 1), jnp.float32)
        for k in range(3):
            mval = jnp.min(d, axis=1, keepdims=True)
            cur = jnp.min(jnp.where(d == mval, lane, M), axis=1,
                          keepdims=True)
            d = jnp.where(lane == cur, _BIG, d)
            wk = 1.0 / (mval + 1e-10)
            Wm = Wm + wk * (lane == cur).astype(jnp.float32)
            wsum = wsum + wk
        Wm = Wm / wsum
        interp = jnp.dot(Wm, cf_ref[0], preferred_element_type=jnp.float32)
        h = jnp.concatenate([interp, ff_ref[0]], axis=1)
        for li, (W, b) in enumerate(wl):
            h = jnp.dot(h, W[...], preferred_element_type=jnp.float32) + b[...]
            if li < n_relu:
                h = jnp.maximum(h, 0.0)
        out_ref[0] = h

    return pl.pallas_call(
        kern,
        grid=(Bc, nblk),
        in_specs=[
            pl.BlockSpec((1, qb, 3), lambda i, j: (i, j, 0)),
            pl.BlockSpec((1, 3, M), lambda i, j: (i, 0, 0)),
            pl.BlockSpec((1, M, F), lambda i, j: (i, 0, 0)),
            pl.BlockSpec((1, qb, S), lambda i, j: (i, j, 0)),
        ] + wspecs,
        out_specs=pl.BlockSpec((1, qb, 3), lambda i, j: (i, j, 0)),
        out_shape=jax.ShapeDtypeStruct((Bc, N, 3), jnp.float32),
        compiler_params=_cparams(2),
    )(fine_xyz, coarse_xyz_t, coarse_feat, fine_feat, *wargs)


def kernel(points1, points2, features1, features2, params):
    P = params
    f1t = features1.transpose(0, 2, 1)
    f2t = features2.transpose(0, 2, 1)
    pts = jnp.concatenate([points1, points2], axis=0)
    ft = jnp.concatenate([f1t, f2t], axis=0)

    nx_t = _fps(pts, 256)
    nx = nx_t.transpose(0, 2, 1)
    table1 = jnp.concatenate([pts, ft], axis=-1)
    f_sc1 = _group(nx, pts.transpose(0, 2, 1), table1, P['sc1'],
                   ns=16, r2=1.0, qb=128)
    B = points1.shape[0]
    p12, p22 = nx[:B], nx[B:]
    p12t, p22t = nx_t[:B], nx_t[B:]
    f12, f22 = f_sc1[:B], f_sc1[B:]

    table_fe = jnp.concatenate([p22, f22], axis=-1)
    emb = _group(p12, p22t, table_fe, P['fe'],
                 ns=64, r2=None, qb=128, self_feat=f12)

    p13t = _fps(p12, 64)
    p13 = p13t.transpose(0, 2, 1)
    table2 = jnp.concatenate([p12, emb], axis=-1)
    f13 = _group(p13, p12t, table2, P['sc2'], ns=8, r2=4.0, qb=64)

    p14t = _fps(p13, 16)
    p14 = p14t.transpose(0, 2, 1)
    table3 = jnp.concatenate([p13, f13], axis=-1)
    f14 = _group(p14, p13t, table3, P['sc3'], ns=8, r2=16.0, qb=16)

    t_up1 = jnp.concatenate([p14, f14], axis=-1)
    g1 = _group(p13, p14t, t_up1, [], ns=8, r2=None, qb=64)
    nf13 = _mlp(jnp.concatenate([g1, f13], axis=-1), P['up1_mlp2'])

    t_up2 = jnp.concatenate([p13, nf13], axis=-1)
    g2 = _group(p12, p13t, t_up2, P['up2_mlp1'], ns=8, r2=None, qb=256)
    skip_t = jnp.concatenate([f12, emb], axis=-1)
    nf12 = _mlp(jnp.concatenate([g2, skip_t], axis=-1), P['up2_mlp2'])

    out = _fp_cls(points1, p12t, nf12, f1t, P['fp'], P['cls'])
    return out.transpose(0, 2, 1)


# speedup vs baseline: 1.2491x; 1.2491x over previous
import functools

import jax
import jax.numpy as jnp
from jax.experimental import pallas as pl
from jax.experimental.pallas import tpu as pltpu

_BIG = 1e10


def _cparams(n):
    return pltpu.CompilerParams(dimension_semantics=("parallel",) * n)


def _iota2(shape, dim):
    return jax.lax.broadcasted_iota(jnp.int32, shape, dim)


def _fps(xyz, npoint):
    Bc, N, _ = xyz.shape
    C = 128 if N >= 128 else N
    R = N // C
    planes = xyz.transpose(0, 2, 1).reshape(Bc, 3, R, C)

    def kern(p_ref, out_ref):
        P = p_ref[...]
        flat = (_iota2((Bc, R, C), 1) * C + _iota2((Bc, R, C), 2))
        lane = _iota2((Bc, 3, npoint), 2)

        def red(x, op):
            return op(op(x, axis=3, keepdims=True), axis=2, keepdims=True)

        def step(t, carry):
            dists, far, CO = carry
            sel = (flat == far)[:, None, :, :]
            cent = red(jnp.where(sel, P, 0.0), jnp.sum)
            CO = jnp.where(lane == t, cent[:, :, :, 0], CO)
            dd = P - cent
            dd = dd * dd
            d = dd[:, 0] + dd[:, 1] + dd[:, 2]
            dists = jnp.minimum(dists, d)
            m = jnp.max(jnp.max(dists, axis=2, keepdims=True), axis=1,
                        keepdims=True)
            far = jnp.min(jnp.min(jnp.where(dists == m, flat, N), axis=2,
                                  keepdims=True), axis=1, keepdims=True)
            return dists, far, CO

        init = (
            jnp.full((Bc, R, C), _BIG, jnp.float32),
            jnp.zeros((Bc, 1, 1), jnp.int32),
            jnp.zeros((Bc, 3, npoint), jnp.float32),
        )
        _, _, CO = jax.lax.fori_loop(0, npoint, step, init)
        out_ref[...] = CO

    return pl.pallas_call(
        kern,
        grid=(1,),
        in_specs=[pl.BlockSpec((Bc, 3, R, C), lambda i: (0, 0, 0, 0))],
        out_specs=pl.BlockSpec((Bc, 3, npoint), lambda i: (0, 0, 0)),
        out_shape=jax.ShapeDtypeStruct((Bc, 3, npoint), jnp.float32),
    )(planes)


def _group(q_xyz, cand_xyz_t, table, ws, ns, r2, qb, self_feat=None):
    Bc, nq, _ = q_xyz.shape
    nc, Ft = table.shape[1], table.shape[2]
    nblk = nq // qb
    S = 0 if self_feat is None else self_feat.shape[2]
    gw = 3 + S + (Ft - 3)
    outF = ws[-1][0].shape[1] if ws else gw

    wargs = []
    wspecs = []
    for (W, b) in ws:
        wargs += [W, b.reshape(1, -1)]
        wspecs += [
            pl.BlockSpec(W.shape, lambda i, j: (0, 0)),
            pl.BlockSpec((1, b.shape[0]), lambda i, j: (0, 0)),
        ]
    sargs = [] if self_feat is None else [self_feat]
    sspecs = [] if self_feat is None else [
        pl.BlockSpec((1, qb, S), lambda i, j: (i, j, 0))
    ]

    def kern(q_ref, cxt_ref, tab_ref, *rest):
        out_ref = rest[-1]
        rest = rest[:-1]
        self_blk = None
        if self_feat is not None:
            self_blk = rest[0][0]
            rest = rest[1:]
        qx = q_ref[0]
        dx = qx[:, 0:1] - cxt_ref[0, 0:1, :]
        dy = qx[:, 1:2] - cxt_ref[0, 1:2, :]
        dz = qx[:, 2:3] - cxt_ref[0, 2:3, :]
        d = dx * dx + dy * dy + dz * dz
        lane = _iota2((qb, nc), 1)

        ohs = []
        if r2 is not None:
            mi = jnp.where(d <= r2, lane, nc)
            first = None
            for k in range(ns):
                cur = jnp.min(mi, axis=1, keepdims=True)
                mi = jnp.where(mi == cur, nc, mi)
                if k == 0:
                    first = jnp.where(cur == nc, 0, cur)
                    idx = first
                else:
                    idx = jnp.where(cur == nc, first, cur)
                ohs.append((lane == idx).astype(jnp.float32))
        else:
            for k in range(ns):
                mval = jnp.min(d, axis=1, keepdims=True)
                cur = jnp.min(jnp.where(d == mval, lane, nc), axis=1,
                              keepdims=True)
                d = jnp.where(lane == cur, _BIG, d)
                ohs.append((lane == cur).astype(jnp.float32))

        tab = tab_ref[0]
        if nc <= 512:
            oh = jnp.concatenate(ohs, axis=0)
            rows = jnp.dot(oh, tab, preferred_element_type=jnp.float32)
        else:
            rows = jnp.concatenate(
                [jnp.dot(o, tab, preferred_element_type=jnp.float32)
                 for o in ohs], axis=0)

        qxt = jnp.concatenate([qx] * ns, axis=0)
        parts = [rows[:, 0:3] - qxt]
        if self_blk is not None:
            parts.append(jnp.concatenate([self_blk] * ns, axis=0))
        parts.append(rows[:, 3:])
        h = jnp.concatenate(parts, axis=1)
        for (W, b) in zip(rest[0::2], rest[1::2]):
            h = jnp.maximum(
                jnp.dot(h, W[...], preferred_element_type=jnp.float32)
                + b[...], 0.0)
        h = h.reshape(ns, qb, outF)
        out_ref[0] = jnp.max(h, axis=0)

    return pl.pallas_call(
        kern,
        grid=(Bc, nblk),
        in_specs=[
            pl.BlockSpec((1, qb, 3), lambda i, j: (i, j, 0)),
            pl.BlockSpec((1, 3, nc), lambda i, j: (i, 0, 0)),
            pl.BlockSpec((1, nc, Ft), lambda i, j: (i, 0, 0)),
        ] + sspecs + wspecs,
        out_specs=pl.BlockSpec((1, qb, outF), lambda i, j: (i, j, 0)),
        out_shape=jax.ShapeDtypeStruct((Bc, nq, outF), jnp.float32),
        compiler_params=_cparams(2),
    )(q_xyz, cand_xyz_t, table, *sargs, *wargs)


def _mlp(x, ws, relu_last=True):
    Bc, rows, _ = x.shape
    outF = ws[-1][0].shape[1]
    wargs = []
    wspecs = []
    for (W, b) in ws:
        wargs += [W, b.reshape(1, -1)]
        wspecs += [
            pl.BlockSpec(W.shape, lambda i: (0, 0)),
            pl.BlockSpec((1, b.shape[0]), lambda i: (0, 0)),
        ]

    def kern(x_ref, *rest):
        out_ref = rest[-1]
        h = x_ref[0]
        wl = list(zip(rest[0:-1:2], rest[1:-1:2]))
        for li, (W, b) in enumerate(wl):
            h = jnp.dot(h, W[...], preferred_element_type=jnp.float32) + b[...]
            if relu_last or li < len(wl) - 1:
                h = jnp.maximum(h, 0.0)
        out_ref[0] = h

    return pl.pallas_call(
        kern,
        grid=(Bc,),
        in_specs=[pl.BlockSpec((1, rows, x.shape[2]), lambda i: (i, 0, 0))]
        + wspecs,
        out_specs=pl.BlockSpec((1, rows, outF), lambda i: (i, 0, 0)),
        out_shape=jax.ShapeDtypeStruct((Bc, rows, outF), jnp.float32),
        compiler_params=_cparams(1),
    )(x, *wargs)


def _fp_cls(fine_xyz, coarse_xyz_t, coarse_feat, fine_feat, fp_ws, cls_ws):
    Bc, N, _ = fine_xyz.shape
    M, F = coarse_feat.shape[1], coarse_feat.shape[2]
    S = fine_feat.shape[2]
    qb = 2048
    nblk = N // qb
    layers = list(fp_ws) + list(cls_ws)
    n_relu = len(layers) - 1
    wargs = []
    wspecs = []
    for (W, b) in layers:
        wargs += [W, b.reshape(1, -1)]
        wspecs += [
            pl.BlockSpec(W.shape, lambda i, j: (0, 0)),
            pl.BlockSpec((1, b.shape[0]), lambda i, j: (0, 0)),
        ]

    def kern(fx_ref, cxt_ref, cf_ref, ff_ref, *rest):
        out_ref = rest[-1]
        wl = list(zip(rest[0:-1:2], rest[1:-1:2]))
        qx = fx_ref[0]
        dx = qx[:, 0:1] - cxt_ref[0, 0:1, :]
        dy = qx[:, 1:2] - cxt_ref[0, 1:2, :]
        dz = qx[:, 2:3] - cxt_ref[0, 2:3, :]
        d = dx * dx + dy * dy + dz * dz
        lane = _iota2((qb, M), 1)
        Wm = jnp.zeros((qb, M), jnp.float32)
        wsum = jnp.zeros((qb, 1), jnp.float32)
        for k in range(3):
            mval = jnp.min(d, axis=1, keepdims=True)
            cur = jnp.min(jnp.where(d == mval, lane, M), axis=1,
                          keepdims=True)
            d = jnp.where(lane == cur, _BIG, d)
            wk = 1.0 / (mval + 1e-10)
            Wm = Wm + wk * (lane == cur).astype(jnp.float32)
            wsum = wsum + wk
        Wm = Wm / wsum
        interp = jnp.dot(Wm, cf_ref[0], preferred_element_type=jnp.float32)
        h = jnp.concatenate([interp, ff_ref[0]], axis=1)
        for li, (W, b) in enumerate(wl):
            h = jnp.dot(h, W[...], preferred_element_type=jnp.float32) + b[...]
            if li < n_relu:
                h = jnp.maximum(h, 0.0)
        out_ref[0] = h

    return pl.pallas_call(
        kern,
        grid=(Bc, nblk),
        in_specs=[
            pl.BlockSpec((1, qb, 3), lambda i, j: (i, j, 0)),
            pl.BlockSpec((1, 3, M), lambda i, j: (i, 0, 0)),
            pl.BlockSpec((1, M, F), lambda i, j: (i, 0, 0)),
            pl.BlockSpec((1, qb, S), lambda i, j: (i, j, 0)),
        ] + wspecs,
        out_specs=pl.BlockSpec((1, qb, 3), lambda i, j: (i, j, 0)),
        out_shape=jax.ShapeDtypeStruct((Bc, N, 3), jnp.float32),
        compiler_params=_cparams(2),
    )(fine_xyz, coarse_xyz_t, coarse_feat, fine_feat, *wargs)


def kernel(points1, points2, features1, features2, params):
    P = params
    f1t = features1.transpose(0, 2, 1)
    f2t = features2.transpose(0, 2, 1)
    pts = jnp.concatenate([points1, points2], axis=0)
    ft = jnp.concatenate([f1t, f2t], axis=0)

    nx_t = _fps(pts, 256)
    nx = nx_t.transpose(0, 2, 1)
    table1 = jnp.concatenate([pts, ft], axis=-1)
    f_sc1 = _group(nx, pts.transpose(0, 2, 1), table1, P['sc1'],
                   ns=16, r2=1.0, qb=128)
    B = points1.shape[0]
    p12, p22 = nx[:B], nx[B:]
    p12t, p22t = nx_t[:B], nx_t[B:]
    f12, f22 = f_sc1[:B], f_sc1[B:]

    table_fe = jnp.concatenate([p22, f22], axis=-1)
    emb = _group(p12, p22t, table_fe, P['fe'],
                 ns=64, r2=None, qb=128, self_feat=f12)

    p13t = _fps(p12, 64)
    p13 = p13t.transpose(0, 2, 1)
    table2 = jnp.concatenate([p12, emb], axis=-1)
    f13 = _group(p13, p12t, table2, P['sc2'], ns=8, r2=4.0, qb=64)

    p14t = _fps(p13, 16)
    p14 = p14t.transpose(0, 2, 1)
    table3 = jnp.concatenate([p13, f13], axis=-1)
    f14 = _group(p14, p13t, table3, P['sc3'], ns=8, r2=16.0, qb=16)

    t_up1 = jnp.concatenate([p14, f14], axis=-1)
    g1 = _group(p13, p14t, t_up1, [], ns=8, r2=None, qb=64)
    nf13 = _mlp(jnp.concatenate([g1, f13], axis=-1), P['up1_mlp2'])

    t_up2 = jnp.concatenate([p13, nf13], axis=-1)
    g2 = _group(p12, p13t, t_up2, P['up2_mlp1'], ns=8, r2=None, qb=256)
    skip_t = jnp.concatenate([f12, emb], axis=-1)
    nf12 = _mlp(jnp.concatenate([g2, skip_t], axis=-1), P['up2_mlp2'])

    out = _fp_cls(points1, p12t, nf12, f1t, P['fp'], P['cls'])
    return out.transpose(0, 2, 1)


# speedup vs baseline: 1.3970x; 1.1184x over previous
import functools

import jax
import jax.numpy as jnp
from jax.experimental import pallas as pl
from jax.experimental.pallas import tpu as pltpu
from jax.experimental.pallas import tpu_sc as plsc

_BIG = 1e10


def _cparams(n):
    return pltpu.CompilerParams(dimension_semantics=("parallel",) * n)


def _iota2(shape, dim):
    return jax.lax.broadcasted_iota(jnp.int32, shape, dim)


def _fps(xyz, npoint):
    Bc, N, _ = xyz.shape
    C = 128 if N >= 128 else N
    R = N // C
    planes = xyz.transpose(0, 2, 1).reshape(Bc, 3, R, C)

    def kern(p_ref, out_ref):
        P = p_ref[...]
        flat = (_iota2((Bc, R, C), 1) * C + _iota2((Bc, R, C), 2))
        lane = _iota2((Bc, 3, npoint), 2)

        def red(x, op):
            return op(op(x, axis=3, keepdims=True), axis=2, keepdims=True)

        def step(t, carry):
            dists, far, CO = carry
            sel = (flat == far)[:, None, :, :]
            cent = red(jnp.where(sel, P, 0.0), jnp.sum)
            CO = jnp.where(lane == t, cent[:, :, :, 0], CO)
            dd = P - cent
            dd = dd * dd
            d = dd[:, 0] + dd[:, 1] + dd[:, 2]
            dists = jnp.minimum(dists, d)
            m = jnp.max(jnp.max(dists, axis=2, keepdims=True), axis=1,
                        keepdims=True)
            far = jnp.min(jnp.min(jnp.where(dists == m, flat, N), axis=2,
                                  keepdims=True), axis=1, keepdims=True)
            return dists, far, CO

        init = (
            jnp.full((Bc, R, C), _BIG, jnp.float32),
            jnp.zeros((Bc, 1, 1), jnp.int32),
            jnp.zeros((Bc, 3, npoint), jnp.float32),
        )
        _, _, CO = jax.lax.fori_loop(0, npoint, step, init)
        out_ref[...] = CO

    return pl.pallas_call(
        kern,
        grid=(1,),
        in_specs=[pl.BlockSpec((Bc, 3, R, C), lambda i: (0, 0, 0, 0))],
        out_specs=pl.BlockSpec((Bc, 3, npoint), lambda i: (0, 0, 0)),
        out_shape=jax.ShapeDtypeStruct((Bc, 3, npoint), jnp.float32),
    )(planes)


def _group(q_xyz, cand_xyz_t, table, ws, ns, r2, qb, self_feat=None):
    Bc, nq, _ = q_xyz.shape
    nc, Ft = table.shape[1], table.shape[2]
    nblk = nq // qb
    S = 0 if self_feat is None else self_feat.shape[2]
    gw = 3 + S + (Ft - 3)
    outF = ws[-1][0].shape[1] if ws else gw

    wargs = []
    wspecs = []
    for (W, b) in ws:
        wargs += [W, b.reshape(1, -1)]
        wspecs += [
            pl.BlockSpec(W.shape, lambda i, j: (0, 0)),
            pl.BlockSpec((1, b.shape[0]), lambda i, j: (0, 0)),
        ]
    sargs = [] if self_feat is None else [self_feat]
    sspecs = [] if self_feat is None else [
        pl.BlockSpec((1, qb, S), lambda i, j: (i, j, 0))
    ]

    def kern(q_ref, cxt_ref, tab_ref, *rest):
        out_ref = rest[-1]
        rest = rest[:-1]
        self_blk = None
        if self_feat is not None:
            self_blk = rest[0][0]
            rest = rest[1:]
        qx = q_ref[0]
        dx = qx[:, 0:1] - cxt_ref[0, 0:1, :]
        dy = qx[:, 1:2] - cxt_ref[0, 1:2, :]
        dz = qx[:, 2:3] - cxt_ref[0, 2:3, :]
        d = dx * dx + dy * dy + dz * dz
        lane = _iota2((qb, nc), 1)

        ohs = []
        if r2 is not None:
            mi = jnp.where(d <= r2, lane, nc)
            first = None
            for k in range(ns):
                cur = jnp.min(mi, axis=1, keepdims=True)
                mi = jnp.where(mi == cur, nc, mi)
                if k == 0:
                    first = jnp.where(cur == nc, 0, cur)
                    idx = first
                else:
                    idx = jnp.where(cur == nc, first, cur)
                ohs.append((lane == idx).astype(jnp.float32))
        else:
            for k in range(ns):
                mval = jnp.min(d, axis=1, keepdims=True)
                cur = jnp.min(jnp.where(d == mval, lane, nc), axis=1,
                              keepdims=True)
                d = jnp.where(lane == cur, _BIG, d)
                ohs.append((lane == cur).astype(jnp.float32))

        tab = tab_ref[0]
        if nc <= 512:
            oh = jnp.concatenate(ohs, axis=0)
            rows = jnp.dot(oh, tab, preferred_element_type=jnp.float32)
        else:
            rows = jnp.concatenate(
                [jnp.dot(o, tab, preferred_element_type=jnp.float32)
                 for o in ohs], axis=0)

        qxt = jnp.concatenate([qx] * ns, axis=0)
        parts = [rows[:, 0:3] - qxt]
        if self_blk is not None:
            parts.append(jnp.concatenate([self_blk] * ns, axis=0))
        parts.append(rows[:, 3:])
        h = jnp.concatenate(parts, axis=1)
        for (W, b) in zip(rest[0::2], rest[1::2]):
            h = jnp.maximum(
                jnp.dot(h, W[...], preferred_element_type=jnp.float32)
                + b[...], 0.0)
        h = h.reshape(ns, qb, outF)
        out_ref[0] = jnp.max(h, axis=0)

    return pl.pallas_call(
        kern,
        grid=(Bc, nblk),
        in_specs=[
            pl.BlockSpec((1, qb, 3), lambda i, j: (i, j, 0)),
            pl.BlockSpec((1, 3, nc), lambda i, j: (i, 0, 0)),
            pl.BlockSpec((1, nc, Ft), lambda i, j: (i, 0, 0)),
        ] + sspecs + wspecs,
        out_specs=pl.BlockSpec((1, qb, outF), lambda i, j: (i, j, 0)),
        out_shape=jax.ShapeDtypeStruct((Bc, nq, outF), jnp.float32),
        compiler_params=_cparams(2),
    )(q_xyz, cand_xyz_t, table, *sargs, *wargs)


def _mlp(x, ws, relu_last=True):
    Bc, rows, _ = x.shape
    outF = ws[-1][0].shape[1]
    wargs = []
    wspecs = []
    for (W, b) in ws:
        wargs += [W, b.reshape(1, -1)]
        wspecs += [
            pl.BlockSpec(W.shape, lambda i: (0, 0)),
            pl.BlockSpec((1, b.shape[0]), lambda i: (0, 0)),
        ]

    def kern(x_ref, *rest):
        out_ref = rest[-1]
        h = x_ref[0]
        wl = list(zip(rest[0:-1:2], rest[1:-1:2]))
        for li, (W, b) in enumerate(wl):
            h = jnp.dot(h, W[...], preferred_element_type=jnp.float32) + b[...]
            if relu_last or li < len(wl) - 1:
                h = jnp.maximum(h, 0.0)
        out_ref[0] = h

    return pl.pallas_call(
        kern,
        grid=(Bc,),
        in_specs=[pl.BlockSpec((1, rows, x.shape[2]), lambda i: (i, 0, 0))]
        + wspecs,
        out_specs=pl.BlockSpec((1, rows, outF), lambda i: (i, 0, 0)),
        out_shape=jax.ShapeDtypeStruct((Bc, rows, outF), jnp.float32),
        compiler_params=_cparams(1),
    )(x, *wargs)


def _fp_cls(fine_xyz, coarse_xyz_t, coarse_feat, fine_feat, fp_ws, cls_ws):
    Bc, N, _ = fine_xyz.shape
    M, F = coarse_feat.shape[1], coarse_feat.shape[2]
    S = fine_feat.shape[2]
    qb = 2048
    nblk = N // qb
    layers = list(fp_ws) + list(cls_ws)
    n_relu = len(layers) - 1
    wargs = []
    wspecs = []
    for (W, b) in layers:
        wargs += [W, b.reshape(1, -1)]
        wspecs += [
            pl.BlockSpec(W.shape, lambda i, j: (0, 0)),
            pl.BlockSpec((1, b.shape[0]), lambda i, j: (0, 0)),
        ]

    def kern(fx_ref, cxt_ref, cf_ref, ff_ref, *rest):
        out_ref = rest[-1]
        wl = list(zip(rest[0:-1:2], rest[1:-1:2]))
        qx = fx_ref[0]
        dx = qx[:, 0:1] - cxt_ref[0, 0:1, :]
        dy = qx[:, 1:2] - cxt_ref[0, 1:2, :]
        dz = qx[:, 2:3] - cxt_ref[0, 2:3, :]
        d = dx * dx + dy * dy + dz * dz
        lane = _iota2((qb, M), 1)
        Wm = jnp.zeros((qb, M), jnp.float32)
        wsum = jnp.zeros((qb, 1), jnp.float32)
        for k in range(3):
            mval = jnp.min(d, axis=1, keepdims=True)
            cur = jnp.min(jnp.where(d == mval, lane, M), axis=1,
                          keepdims=True)
            d = jnp.where(lane == cur, _BIG, d)
            wk = 1.0 / (mval + 1e-10)
            Wm = Wm + wk * (lane == cur).astype(jnp.float32)
            wsum = wsum + wk
        Wm = Wm / wsum
        interp = jnp.dot(Wm, cf_ref[0], preferred_element_type=jnp.float32)
        h = jnp.concatenate([interp, ff_ref[0]], axis=1)
        for li, (W, b) in enumerate(wl):
            h = jnp.dot(h, W[...], preferred_element_type=jnp.float32) + b[...]
            if li < n_relu:
                h = jnp.maximum(h, 0.0)
        out_ref[0] = h

    return pl.pallas_call(
        kern,
        grid=(Bc, nblk),
        in_specs=[
            pl.BlockSpec((1, qb, 3), lambda i, j: (i, j, 0)),
            pl.BlockSpec((1, 3, M), lambda i, j: (i, 0, 0)),
            pl.BlockSpec((1, M, F), lambda i, j: (i, 0, 0)),
            pl.BlockSpec((1, qb, S), lambda i, j: (i, j, 0)),
        ] + wspecs,
        out_specs=pl.BlockSpec((1, qb, 3), lambda i, j: (i, j, 0)),
        out_shape=jax.ShapeDtypeStruct((Bc, N, 3), jnp.float32),
        compiler_params=_cparams(2),
    )(fine_xyz, coarse_xyz_t, coarse_feat, fine_feat, *wargs)


def _ball_select(q_xyz, cand_xyz_t, ns, r2, qb):
    Bc, nq, _ = q_xyz.shape
    nc = cand_xyz_t.shape[2]
    nblk = nq // qb

    def kern(q_ref, cxt_ref, out_ref):
        b = pl.program_id(0)
        qx = q_ref[0]
        dx = qx[:, 0:1] - cxt_ref[0, 0:1, :]
        dy = qx[:, 1:2] - cxt_ref[0, 1:2, :]
        dz = qx[:, 2:3] - cxt_ref[0, 2:3, :]
        d = dx * dx + dy * dy + dz * dz
        lane = _iota2((qb, nc), 1)
        mi = jnp.where(d <= r2, lane, nc)
        first = None
        for k in range(ns):
            cur = jnp.min(mi, axis=1, keepdims=True)
            mi = jnp.where(mi == cur, nc, mi)
            if k == 0:
                first = jnp.where(cur == nc, 0, cur)
                idx = first
            else:
                idx = jnp.where(cur == nc, first, cur)
            out_ref[0, 0, :, k : k + 1] = idx + b * nc

    return pl.pallas_call(
        kern,
        grid=(Bc, nblk),
        in_specs=[
            pl.BlockSpec((1, qb, 3), lambda i, j: (i, j, 0)),
            pl.BlockSpec((1, 3, nc), lambda i, j: (i, 0, 0)),
        ],
        out_specs=pl.BlockSpec((1, 1, qb, ns), lambda i, j: (i, j, 0, 0)),
        out_shape=jax.ShapeDtypeStruct((Bc, nblk, qb, ns), jnp.int32),
    )(q_xyz, cand_xyz_t)


def _sc_gather(table, idx):
    V, D = table.shape
    nrow = idx.shape[0]
    NW = 32
    b_per_w = nrow // NW
    nch = b_per_w // 128
    WAVE = 4
    idx2 = idx.reshape(NW * nch, 128)
    mesh = plsc.VectorSubcoreMesh(core_axis_name="c", subcore_axis_name="s")

    @functools.partial(
        pl.kernel,
        mesh=mesh,
        out_type=jax.ShapeDtypeStruct((nrow, D), jnp.float32),
        scratch_types=[
            pltpu.VMEM((nch, 128), jnp.int32),
            pltpu.VMEM((WAVE * 128, D), jnp.float32),
            pltpu.SemaphoreType.DMA,
        ],
    )
    def k(tab_hbm, idx_hbm, out_hbm, idx_v, rows_v, sem):
        wid = jax.lax.axis_index("s") * 2 + jax.lax.axis_index("c")
        pltpu.sync_copy(idx_hbm.at[pl.ds(wid * nch, nch)], idx_v)
        for w in range(nch // WAVE):
            cps = [
                pltpu.async_copy(
                    tab_hbm.at[idx_v.at[w * WAVE + j]],
                    rows_v.at[pl.ds(j * 128, 128)],
                    sem,
                )
                for j in range(WAVE)
            ]
            for cp in cps:
                cp.wait()
            pltpu.sync_copy(
                rows_v,
                out_hbm.at[pl.ds(wid * b_per_w + w * WAVE * 128,
                                 WAVE * 128)])

    return k(table, idx2)


def _post_group_mlp(rows, q_xyz, ws, ns, fw):
    Bc, nblk, qbns, Dp = rows.shape
    qb = qbns // ns
    nq = nblk * qb
    outF = ws[-1][0].shape[1]
    wargs = []
    wspecs = []
    for (W, b) in ws:
        wargs += [W, b.reshape(1, -1)]
        wspecs += [
            pl.BlockSpec(W.shape, lambda i, j: (0, 0)),
            pl.BlockSpec((1, b.shape[0]), lambda i, j: (0, 0)),
        ]

    def kern(r_ref, q_ref, *rest):
        out_ref = rest[-1]
        rows_b = r_ref[0, 0]
        qx = q_ref[0]
        qxrep = jnp.broadcast_to(qx[:, None, :], (qb, ns, 3)).reshape(
            qb * ns, 3)
        h = jnp.concatenate(
            [rows_b[:, 0:3] - qxrep, rows_b[:, 3:fw]], axis=1)
        for (W, b) in zip(rest[0:-1:2], rest[1:-1:2]):
            h = jnp.maximum(
                jnp.dot(h, W[...], preferred_element_type=jnp.float32)
                + b[...], 0.0)
        out_ref[0] = jnp.max(h.reshape(qb, ns, outF), axis=1)

    return pl.pallas_call(
        kern,
        grid=(Bc, nblk),
        in_specs=[
            pl.BlockSpec((1, 1, qbns, Dp), lambda i, j: (i, j, 0, 0)),
            pl.BlockSpec((1, qb, 3), lambda i, j: (i, j, 0)),
        ] + wspecs,
        out_specs=pl.BlockSpec((1, qb, outF), lambda i, j: (i, j, 0)),
        out_shape=jax.ShapeDtypeStruct((Bc, nq, outF), jnp.float32),
    )(rows, q_xyz, *wargs)


def kernel(points1, points2, features1, features2, params):
    P = params
    f1t = features1.transpose(0, 2, 1)
    f2t = features2.transpose(0, 2, 1)
    pts = jnp.concatenate([points1, points2], axis=0)
    ft = jnp.concatenate([f1t, f2t], axis=0)

    nx_t = _fps(pts, 256)
    nx = nx_t.transpose(0, 2, 1)
    B2, N = pts.shape[0], pts.shape[1]
    qb1, ns1 = 128, 16
    table1 = jnp.concatenate(
        [pts, ft, jnp.zeros((B2, N, 61), jnp.float32)], axis=-1)
    idx1 = _ball_select(nx, pts.transpose(0, 2, 1), ns=ns1, r2=1.0, qb=qb1)
    rows1 = _sc_gather(table1.reshape(B2 * N, 128), idx1.reshape(-1))
    rows1 = rows1.reshape(B2, 256 // qb1, qb1 * ns1, 128)
    f_sc1 = _post_group_mlp(rows1, nx, P['sc1'], ns=ns1, fw=67)
    B = points1.shape[0]
    p12, p22 = nx[:B], nx[B:]
    p12t, p22t = nx_t[:B], nx_t[B:]
    f12, f22 = f_sc1[:B], f_sc1[B:]

    table_fe = jnp.concatenate([p22, f22], axis=-1)
    emb = _group(p12, p22t, table_fe, P['fe'],
                 ns=64, r2=None, qb=128, self_feat=f12)

    p13t = _fps(p12, 64)
    p13 = p13t.transpose(0, 2, 1)
    table2 = jnp.concatenate([p12, emb], axis=-1)
    f13 = _group(p13, p12t, table2, P['sc2'], ns=8, r2=4.0, qb=64)

    p14t = _fps(p13, 16)
    p14 = p14t.transpose(0, 2, 1)
    table3 = jnp.concatenate([p13, f13], axis=-1)
    f14 = _group(p14, p13t, table3, P['sc3'], ns=8, r2=16.0, qb=16)

    t_up1 = jnp.concatenate([p14, f14], axis=-1)
    g1 = _group(p13, p14t, t_up1, [], ns=8, r2=None, qb=64)
    nf13 = _mlp(jnp.concatenate([g1, f13], axis=-1), P['up1_mlp2'])

    t_up2 = jnp.concatenate([p13, nf13], axis=-1)
    g2 = _group(p12, p13t, t_up2, P['up2_mlp1'], ns=8, r2=None, qb=256)
    skip_t = jnp.concatenate([f12, emb], axis=-1)
    nf12 = _mlp(jnp.concatenate([g2, skip_t], axis=-1), P['up2_mlp2'])

    out = _fp_cls(points1, p12t, nf12, f1t, P['fp'], P['cls'])
    return out.transpose(0, 2, 1)


# speedup vs baseline: 1.4294x; 1.0232x over previous
import functools

import jax
import jax.numpy as jnp
from jax.experimental import pallas as pl
from jax.experimental.pallas import tpu as pltpu
from jax.experimental.pallas import tpu_sc as plsc

_BIG = 1e10


def _cparams(n):
    return pltpu.CompilerParams(dimension_semantics=("parallel",) * n)


def _iota2(shape, dim):
    return jax.lax.broadcasted_iota(jnp.int32, shape, dim)


def _fps(xyz, npoint):
    Bc, N, _ = xyz.shape
    C = 128 if N >= 128 else N
    R = N // C
    planes = xyz.transpose(0, 2, 1).reshape(Bc, 3, R, C)

    def kern(p_ref, out_ref):
        P = p_ref[...]
        flat = (_iota2((Bc, R, C), 1) * C + _iota2((Bc, R, C), 2))
        lane = _iota2((Bc, 3, npoint), 2)

        def red(x, op):
            return op(op(x, axis=3, keepdims=True), axis=2, keepdims=True)

        def step(t, carry):
            dists, far, CO = carry
            sel = (flat == far)[:, None, :, :]
            cent = red(jnp.where(sel, P, 0.0), jnp.sum)
            CO = jnp.where(lane == t, cent[:, :, :, 0], CO)
            dd = P - cent
            dd = dd * dd
            d = dd[:, 0] + dd[:, 1] + dd[:, 2]
            dists = jnp.minimum(dists, d)
            m = jnp.max(jnp.max(dists, axis=2, keepdims=True), axis=1,
                        keepdims=True)
            far = jnp.min(jnp.min(jnp.where(dists == m, flat, N), axis=2,
                                  keepdims=True), axis=1, keepdims=True)
            return dists, far, CO

        init = (
            jnp.full((Bc, R, C), _BIG, jnp.float32),
            jnp.zeros((Bc, 1, 1), jnp.int32),
            jnp.zeros((Bc, 3, npoint), jnp.float32),
        )
        _, _, CO = jax.lax.fori_loop(0, npoint, step, init)
        out_ref[...] = CO

    return pl.pallas_call(
        kern,
        grid=(1,),
        in_specs=[pl.BlockSpec((Bc, 3, R, C), lambda i: (0, 0, 0, 0))],
        out_specs=pl.BlockSpec((Bc, 3, npoint), lambda i: (0, 0, 0)),
        out_shape=jax.ShapeDtypeStruct((Bc, 3, npoint), jnp.float32),
    )(planes)


def _group(q_xyz, cand_xyz_t, table, ws, ns, r2, qb, self_feat=None):
    Bc, nq, _ = q_xyz.shape
    nc, Ft = table.shape[1], table.shape[2]
    nblk = nq // qb
    S = 0 if self_feat is None else self_feat.shape[2]
    gw = 3 + S + (Ft - 3)
    outF = ws[-1][0].shape[1] if ws else gw

    wargs = []
    wspecs = []
    for (W, b) in ws:
        wargs += [W, b.reshape(1, -1)]
        wspecs += [
            pl.BlockSpec(W.shape, lambda i, j: (0, 0)),
            pl.BlockSpec((1, b.shape[0]), lambda i, j: (0, 0)),
        ]
    sargs = [] if self_feat is None else [self_feat]
    sspecs = [] if self_feat is None else [
        pl.BlockSpec((1, qb, S), lambda i, j: (i, j, 0))
    ]

    def kern(q_ref, cxt_ref, tab_ref, *rest):
        out_ref = rest[-1]
        rest = rest[:-1]
        self_blk = None
        if self_feat is not None:
            self_blk = rest[0][0]
            rest = rest[1:]
        qx = q_ref[0]
        dx = qx[:, 0:1] - cxt_ref[0, 0:1, :]
        dy = qx[:, 1:2] - cxt_ref[0, 1:2, :]
        dz = qx[:, 2:3] - cxt_ref[0, 2:3, :]
        d = dx * dx + dy * dy + dz * dz
        lane = _iota2((qb, nc), 1)

        ohs = []
        if r2 is not None:
            mi = jnp.where(d <= r2, lane, nc)
            first = None
            for k in range(ns):
                cur = jnp.min(mi, axis=1, keepdims=True)
                mi = jnp.where(mi == cur, nc, mi)
                if k == 0:
                    first = jnp.where(cur == nc, 0, cur)
                    idx = first
                else:
                    idx = jnp.where(cur == nc, first, cur)
                ohs.append((lane == idx).astype(jnp.float32))
        else:
            for k in range(ns):
                mval = jnp.min(d, axis=1, keepdims=True)
                cur = jnp.min(jnp.where(d == mval, lane, nc), axis=1,
                              keepdims=True)
                d = jnp.where(lane == cur, _BIG, d)
                ohs.append((lane == cur).astype(jnp.float32))

        tab = tab_ref[0]
        if self_blk is not None:
            W1, b1 = rest[0], rest[1]
            Wac = jnp.concatenate([W1[0:3, :], W1[3 + S :, :]], axis=0)
            A = jnp.dot(tab, Wac, preferred_element_type=jnp.float32)
            Bq = (jnp.dot(self_blk, W1[3 : 3 + S, :],
                          preferred_element_type=jnp.float32)
                  - jnp.dot(qx, W1[0:3, :],
                            preferred_element_type=jnp.float32) + b1[...])
            oh = jnp.concatenate(ohs, axis=0)
            h = jnp.maximum(
                jnp.dot(oh, A, preferred_element_type=jnp.float32)
                + jnp.concatenate([Bq] * ns, axis=0), 0.0)
            for (W, b) in zip(rest[2::2], rest[3::2]):
                h = jnp.maximum(
                    jnp.dot(h, W[...], preferred_element_type=jnp.float32)
                    + b[...], 0.0)
        else:
            if nc <= 512:
                oh = jnp.concatenate(ohs, axis=0)
                rows = jnp.dot(oh, tab, preferred_element_type=jnp.float32)
            else:
                rows = jnp.concatenate(
                    [jnp.dot(o, tab, preferred_element_type=jnp.float32)
                     for o in ohs], axis=0)

            qxt = jnp.concatenate([qx] * ns, axis=0)
            parts = [rows[:, 0:3] - qxt, rows[:, 3:]]
            h = jnp.concatenate(parts, axis=1)
            for (W, b) in zip(rest[0::2], rest[1::2]):
                h = jnp.maximum(
                    jnp.dot(h, W[...], preferred_element_type=jnp.float32)
                    + b[...], 0.0)
        h = h.reshape(ns, qb, outF)
        out_ref[0] = jnp.max(h, axis=0)

    return pl.pallas_call(
        kern,
        grid=(Bc, nblk),
        in_specs=[
            pl.BlockSpec((1, qb, 3), lambda i, j: (i, j, 0)),
            pl.BlockSpec((1, 3, nc), lambda i, j: (i, 0, 0)),
            pl.BlockSpec((1, nc, Ft), lambda i, j: (i, 0, 0)),
        ] + sspecs + wspecs,
        out_specs=pl.BlockSpec((1, qb, outF), lambda i, j: (i, j, 0)),
        out_shape=jax.ShapeDtypeStruct((Bc, nq, outF), jnp.float32),
        compiler_params=_cparams(2),
    )(q_xyz, cand_xyz_t, table, *sargs, *wargs)


def _mlp(x, ws, relu_last=True):
    Bc, rows, _ = x.shape
    outF = ws[-1][0].shape[1]
    wargs = []
    wspecs = []
    for (W, b) in ws:
        wargs += [W, b.reshape(1, -1)]
        wspecs += [
            pl.BlockSpec(W.shape, lambda i: (0, 0)),
            pl.BlockSpec((1, b.shape[0]), lambda i: (0, 0)),
        ]

    def kern(x_ref, *rest):
        out_ref = rest[-1]
        h = x_ref[0]
        wl = list(zip(rest[0:-1:2], rest[1:-1:2]))
        for li, (W, b) in enumerate(wl):
            h = jnp.dot(h, W[...], preferred_element_type=jnp.float32) + b[...]
            if relu_last or li < len(wl) - 1:
                h = jnp.maximum(h, 0.0)
        out_ref[0] = h

    return pl.pallas_call(
        kern,
        grid=(Bc,),
        in_specs=[pl.BlockSpec((1, rows, x.shape[2]), lambda i: (i, 0, 0))]
        + wspecs,
        out_specs=pl.BlockSpec((1, rows, outF), lambda i: (i, 0, 0)),
        out_shape=jax.ShapeDtypeStruct((Bc, rows, outF), jnp.float32),
        compiler_params=_cparams(1),
    )(x, *wargs)


def _fp_cls(fine_xyz, coarse_xyz_t, coarse_feat, fine_feat, fp_ws, cls_ws):
    Bc, N, _ = fine_xyz.shape
    M, F = coarse_feat.shape[1], coarse_feat.shape[2]
    S = fine_feat.shape[2]
    qb = 2048
    nblk = N // qb
    layers = list(fp_ws) + list(cls_ws)
    n_relu = len(layers) - 1
    wargs = []
    wspecs = []
    for (W, b) in layers:
        wargs += [W, b.reshape(1, -1)]
        wspecs += [
            pl.BlockSpec(W.shape, lambda i, j: (0, 0)),
            pl.BlockSpec((1, b.shape[0]), lambda i, j: (0, 0)),
        ]

    def kern(fx_ref, cxt_ref, cf_ref, ff_ref, *rest):
        out_ref = rest[-1]
        wl = list(zip(rest[0:-1:2], rest[1:-1:2]))
        qx = fx_ref[0]
        dx = qx[:, 0:1] - cxt_ref[0, 0:1, :]
        dy = qx[:, 1:2] - cxt_ref[0, 1:2, :]
        dz = qx[:, 2:3] - cxt_ref[0, 2:3, :]
        d = dx * dx + dy * dy + dz * dz
        lane = _iota2((qb, M), 1)
        Wm = jnp.zeros((qb, M), jnp.float32)
        wsum = jnp.zeros((qb, 1), jnp.float32)
        for k in range(3):
            mval = jnp.min(d, axis=1, keepdims=True)
            cur = jnp.min(jnp.where(d == mval, lane, M), axis=1,
                          keepdims=True)
            d = jnp.where(lane == cur, _BIG, d)
            wk = 1.0 / (mval + 1e-10)
            Wm = Wm + wk * (lane == cur).astype(jnp.float32)
            wsum = wsum + wk
        Wm = Wm / wsum
        interp = jnp.dot(Wm, cf_ref[0], preferred_element_type=jnp.float32)
        h = jnp.concatenate([interp, ff_ref[0]], axis=1)
        for li, (W, b) in enumerate(wl):
            h = jnp.dot(h, W[...], preferred_element_type=jnp.float32) + b[...]
            if li < n_relu:
                h = jnp.maximum(h, 0.0)
        out_ref[0] = h

    return pl.pallas_call(
        kern,
        grid=(Bc, nblk),
        in_specs=[
            pl.BlockSpec((1, qb, 3), lambda i, j: (i, j, 0)),
            pl.BlockSpec((1, 3, M), lambda i, j: (i, 0, 0)),
            pl.BlockSpec((1, M, F), lambda i, j: (i, 0, 0)),
            pl.BlockSpec((1, qb, S), lambda i, j: (i, j, 0)),
        ] + wspecs,
        out_specs=pl.BlockSpec((1, qb, 3), lambda i, j: (i, j, 0)),
        out_shape=jax.ShapeDtypeStruct((Bc, N, 3), jnp.float32),
        compiler_params=_cparams(2),
    )(fine_xyz, coarse_xyz_t, coarse_feat, fine_feat, *wargs)


def _ball_select(q_xyz, cand_xyz_t, ns, r2, qb):
    Bc, nq, _ = q_xyz.shape
    nc = cand_xyz_t.shape[2]
    nblk = nq // qb

    def kern(q_ref, cxt_ref, out_ref):
        b = pl.program_id(0)
        qx = q_ref[0]
        dx = qx[:, 0:1] - cxt_ref[0, 0:1, :]
        dy = qx[:, 1:2] - cxt_ref[0, 1:2, :]
        dz = qx[:, 2:3] - cxt_ref[0, 2:3, :]
        d = dx * dx + dy * dy + dz * dz
        lane = _iota2((qb, nc), 1)
        mi = jnp.where(d <= r2, lane, nc)
        first = None
        for k in range(ns):
            cur = jnp.min(mi, axis=1, keepdims=True)
            mi = jnp.where(mi == cur, nc, mi)
            if k == 0:
                first = jnp.where(cur == nc, 0, cur)
                idx = first
            else:
                idx = jnp.where(cur == nc, first, cur)
            out_ref[0, 0, :, k : k + 1] = idx + b * nc

    return pl.pallas_call(
        kern,
        grid=(Bc, nblk),
        in_specs=[
            pl.BlockSpec((1, qb, 3), lambda i, j: (i, j, 0)),
            pl.BlockSpec((1, 3, nc), lambda i, j: (i, 0, 0)),
        ],
        out_specs=pl.BlockSpec((1, 1, qb, ns), lambda i, j: (i, j, 0, 0)),
        out_shape=jax.ShapeDtypeStruct((Bc, nblk, qb, ns), jnp.int32),
    )(q_xyz, cand_xyz_t)


def _sc_gather(table, idx):
    V, D = table.shape
    nrow = idx.shape[0]
    NW = 32
    b_per_w = nrow // NW
    nch = b_per_w // 128
    WAVE = 4
    idx2 = idx.reshape(NW * nch, 128)
    mesh = plsc.VectorSubcoreMesh(core_axis_name="c", subcore_axis_name="s")

    @functools.partial(
        pl.kernel,
        mesh=mesh,
        out_type=jax.ShapeDtypeStruct((nrow, D), jnp.float32),
        scratch_types=[
            pltpu.VMEM((nch, 128), jnp.int32),
            pltpu.VMEM((WAVE * 128, D), jnp.float32),
            pltpu.SemaphoreType.DMA,
        ],
    )
    def k(tab_hbm, idx_hbm, out_hbm, idx_v, rows_v, sem):
        wid = jax.lax.axis_index("s") * 2 + jax.lax.axis_index("c")
        pltpu.sync_copy(idx_hbm.at[pl.ds(wid * nch, nch)], idx_v)
        for w in range(nch // WAVE):
            cps = [
                pltpu.async_copy(
                    tab_hbm.at[idx_v.at[w * WAVE + j]],
                    rows_v.at[pl.ds(j * 128, 128)],
                    sem,
                )
                for j in range(WAVE)
            ]
            for cp in cps:
                cp.wait()
            pltpu.sync_copy(
                rows_v,
                out_hbm.at[pl.ds(wid * b_per_w + w * WAVE * 128,
                                 WAVE * 128)])

    return k(table, idx2)


def _post_group_mlp(rows, q_xyz, ws, ns, fw):
    Bc, nblk, qbns, Dp = rows.shape
    qb = qbns // ns
    nq = nblk * qb
    outF = ws[-1][0].shape[1]
    wargs = []
    wspecs = []
    for (W, b) in ws:
        wargs += [W, b.reshape(1, -1)]
        wspecs += [
            pl.BlockSpec(W.shape, lambda i, j: (0, 0)),
            pl.BlockSpec((1, b.shape[0]), lambda i, j: (0, 0)),
        ]

    def kern(r_ref, q_ref, *rest):
        out_ref = rest[-1]
        rows_b = r_ref[0, 0]
        qx = q_ref[0]
        qxrep = jnp.broadcast_to(qx[:, None, :], (qb, ns, 3)).reshape(
            qb * ns, 3)
        h = jnp.concatenate(
            [rows_b[:, 0:3] - qxrep, rows_b[:, 3:fw]], axis=1)
        for (W, b) in zip(rest[0:-1:2], rest[1:-1:2]):
            h = jnp.maximum(
                jnp.dot(h, W[...], preferred_element_type=jnp.float32)
                + b[...], 0.0)
        out_ref[0] = jnp.max(h.reshape(qb, ns, outF), axis=1)

    return pl.pallas_call(
        kern,
        grid=(Bc, nblk),
        in_specs=[
            pl.BlockSpec((1, 1, qbns, Dp), lambda i, j: (i, j, 0, 0)),
            pl.BlockSpec((1, qb, 3), lambda i, j: (i, j, 0)),
        ] + wspecs,
        out_specs=pl.BlockSpec((1, qb, outF), lambda i, j: (i, j, 0)),
        out_shape=jax.ShapeDtypeStruct((Bc, nq, outF), jnp.float32),
    )(rows, q_xyz, *wargs)


def kernel(points1, points2, features1, features2, params):
    P = params
    f1t = features1.transpose(0, 2, 1)
    f2t = features2.transpose(0, 2, 1)
    pts = jnp.concatenate([points1, points2], axis=0)
    ft = jnp.concatenate([f1t, f2t], axis=0)

    nx_t = _fps(pts, 256)
    nx = nx_t.transpose(0, 2, 1)
    B2, N = pts.shape[0], pts.shape[1]
    qb1, ns1 = 128, 16
    table1 = jnp.concatenate(
        [pts, ft, jnp.zeros((B2, N, 61), jnp.float32)], axis=-1)
    idx1 = _ball_select(nx, pts.transpose(0, 2, 1), ns=ns1, r2=1.0, qb=qb1)
    rows1 = _sc_gather(table1.reshape(B2 * N, 128), idx1.reshape(-1))
    rows1 = rows1.reshape(B2, 256 // qb1, qb1 * ns1, 128)
    f_sc1 = _post_group_mlp(rows1, nx, P['sc1'], ns=ns1, fw=67)
    B = points1.shape[0]
    p12, p22 = nx[:B], nx[B:]
    p12t, p22t = nx_t[:B], nx_t[B:]
    f12, f22 = f_sc1[:B], f_sc1[B:]

    table_fe = jnp.concatenate([p22, f22], axis=-1)
    emb = _group(p12, p22t, table_fe, P['fe'],
                 ns=64, r2=None, qb=128, self_feat=f12)

    p13t = _fps(p12, 64)
    p13 = p13t.transpose(0, 2, 1)
    table2 = jnp.concatenate([p12, emb], axis=-1)
    f13 = _group(p13, p12t, table2, P['sc2'], ns=8, r2=4.0, qb=64)

    p14t = _fps(p13, 16)
    p14 = p14t.transpose(0, 2, 1)
    table3 = jnp.concatenate([p13, f13], axis=-1)
    f14 = _group(p14, p13t, table3, P['sc3'], ns=8, r2=16.0, qb=16)

    t_up1 = jnp.concatenate([p14, f14], axis=-1)
    g1 = _group(p13, p14t, t_up1, [], ns=8, r2=None, qb=64)
    nf13 = _mlp(jnp.concatenate([g1, f13], axis=-1), P['up1_mlp2'])

    t_up2 = jnp.concatenate([p13, nf13], axis=-1)
    g2 = _group(p12, p13t, t_up2, P['up2_mlp1'], ns=8, r2=None, qb=256)
    skip_t = jnp.concatenate([f12, emb], axis=-1)
    nf12 = _mlp(jnp.concatenate([g2, skip_t], axis=-1), P['up2_mlp2'])

    out = _fp_cls(points1, p12t, nf12, f1t, P['fp'], P['cls'])
    return out.transpose(0, 2, 1)


# speedup vs baseline: 1.4477x; 1.0129x over previous
import functools

import jax
import jax.numpy as jnp
from jax.experimental import pallas as pl
from jax.experimental.pallas import tpu as pltpu
from jax.experimental.pallas import tpu_sc as plsc

_BIG = 1e10


def _cparams(n):
    return pltpu.CompilerParams(dimension_semantics=("parallel",) * n)


def _iota2(shape, dim):
    return jax.lax.broadcasted_iota(jnp.int32, shape, dim)


def _fps(xyz, npoint):
    Bc, N, _ = xyz.shape
    C = 128 if N >= 128 else N
    R = N // C
    planes = xyz.transpose(0, 2, 1).reshape(Bc, 3, R, C)

    def kern(p_ref, out_ref):
        P = p_ref[...]
        flat = (_iota2((Bc, R, C), 1) * C + _iota2((Bc, R, C), 2))
        lane = _iota2((Bc, 3, npoint), 2)

        def red(x, op):
            return op(op(x, axis=3, keepdims=True), axis=2, keepdims=True)

        def step(t, carry):
            dists, far, CO = carry
            sel = (flat == far)[:, None, :, :]
            cent = red(jnp.where(sel, P, 0.0), jnp.sum)
            CO = jnp.where(lane == t, cent[:, :, :, 0], CO)
            dd = P - cent
            dd = dd * dd
            d = dd[:, 0] + dd[:, 1] + dd[:, 2]
            dists = jnp.minimum(dists, d)
            m = jnp.max(jnp.max(dists, axis=2, keepdims=True), axis=1,
                        keepdims=True)
            far = jnp.min(jnp.min(jnp.where(dists == m, flat, N), axis=2,
                                  keepdims=True), axis=1, keepdims=True)
            return dists, far, CO

        init = (
            jnp.full((Bc, R, C), _BIG, jnp.float32),
            jnp.zeros((Bc, 1, 1), jnp.int32),
            jnp.zeros((Bc, 3, npoint), jnp.float32),
        )
        _, _, CO = jax.lax.fori_loop(0, npoint, step, init)
        out_ref[...] = CO

    return pl.pallas_call(
        kern,
        grid=(1,),
        in_specs=[pl.BlockSpec((Bc, 3, R, C), lambda i: (0, 0, 0, 0))],
        out_specs=pl.BlockSpec((Bc, 3, npoint), lambda i: (0, 0, 0)),
        out_shape=jax.ShapeDtypeStruct((Bc, 3, npoint), jnp.float32),
    )(planes)


def _group(q_xyz, cand_xyz_t, table, ws, ns, r2, qb, self_feat=None):
    Bc, nq, _ = q_xyz.shape
    nc, Ft = table.shape[1], table.shape[2]
    nblk = nq // qb
    S = 0 if self_feat is None else self_feat.shape[2]
    gw = 3 + S + (Ft - 3)
    outF = ws[-1][0].shape[1] if ws else gw

    wargs = []
    wspecs = []
    for (W, b) in ws:
        wargs += [W, b.reshape(1, -1)]
        wspecs += [
            pl.BlockSpec(W.shape, lambda i, j: (0, 0)),
            pl.BlockSpec((1, b.shape[0]), lambda i, j: (0, 0)),
        ]
    sargs = [] if self_feat is None else [self_feat]
    sspecs = [] if self_feat is None else [
        pl.BlockSpec((1, qb, S), lambda i, j: (i, j, 0))
    ]

    def kern(q_ref, cxt_ref, tab_ref, *rest):
        out_ref = rest[-1]
        rest = rest[:-1]
        self_blk = None
        if self_feat is not None:
            self_blk = rest[0][0]
            rest = rest[1:]
        qx = q_ref[0]
        dx = qx[:, 0:1] - cxt_ref[0, 0:1, :]
        dy = qx[:, 1:2] - cxt_ref[0, 1:2, :]
        dz = qx[:, 2:3] - cxt_ref[0, 2:3, :]
        d = dx * dx + dy * dy + dz * dz
        lane = _iota2((qb, nc), 1)

        ohs = []
        if r2 is not None:
            mi = jnp.where(d <= r2, lane, nc)
            first = None
            for k in range(ns):
                cur = jnp.min(mi, axis=1, keepdims=True)
                mi = jnp.where(mi == cur, nc, mi)
                if k == 0:
                    first = jnp.where(cur == nc, 0, cur)
                    idx = first
                else:
                    idx = jnp.where(cur == nc, first, cur)
                ohs.append((lane == idx).astype(jnp.float32))
        else:
            n3 = _iota2((qb, 1, nc), 2)
            R = jnp.zeros((qb, nc), jnp.float32)
            CH = 128 if nc > 128 else nc
            for m0 in range(0, nc, CH):
                dm = d[:, m0 : m0 + CH]
                m3 = _iota2((qb, CH, 1), 1) + m0
                lt = dm[:, :, None] < d[:, None, :]
                eq = dm[:, :, None] == d[:, None, :]
                cmp = lt | (eq & (m3 < n3))
                R = R + jnp.sum(cmp.astype(jnp.float32), axis=1)
            for k in range(ns):
                ohs.append((R == float(k)).astype(jnp.float32))

        tab = tab_ref[0]
        if self_blk is not None:
            W1, b1 = rest[0], rest[1]
            Wac = jnp.concatenate([W1[0:3, :], W1[3 + S :, :]], axis=0)
            A = jnp.dot(tab, Wac, preferred_element_type=jnp.float32)
            Bq = (jnp.dot(self_blk, W1[3 : 3 + S, :],
                          preferred_element_type=jnp.float32)
                  - jnp.dot(qx, W1[0:3, :],
                            preferred_element_type=jnp.float32) + b1[...])
            oh = jnp.concatenate(ohs, axis=0)
            h = jnp.maximum(
                jnp.dot(oh, A, preferred_element_type=jnp.float32)
                + jnp.concatenate([Bq] * ns, axis=0), 0.0)
            for (W, b) in zip(rest[2::2], rest[3::2]):
                h = jnp.maximum(
                    jnp.dot(h, W[...], preferred_element_type=jnp.float32)
                    + b[...], 0.0)
        else:
            if nc <= 512:
                oh = jnp.concatenate(ohs, axis=0)
                rows = jnp.dot(oh, tab, preferred_element_type=jnp.float32)
            else:
                rows = jnp.concatenate(
                    [jnp.dot(o, tab, preferred_element_type=jnp.float32)
                     for o in ohs], axis=0)

            qxt = jnp.concatenate([qx] * ns, axis=0)
            parts = [rows[:, 0:3] - qxt, rows[:, 3:]]
            h = jnp.concatenate(parts, axis=1)
            for (W, b) in zip(rest[0::2], rest[1::2]):
                h = jnp.maximum(
                    jnp.dot(h, W[...], preferred_element_type=jnp.float32)
                    + b[...], 0.0)
        h = h.reshape(ns, qb, outF)
        out_ref[0] = jnp.max(h, axis=0)

    return pl.pallas_call(
        kern,
        grid=(Bc, nblk),
        in_specs=[
            pl.BlockSpec((1, qb, 3), lambda i, j: (i, j, 0)),
            pl.BlockSpec((1, 3, nc), lambda i, j: (i, 0, 0)),
            pl.BlockSpec((1, nc, Ft), lambda i, j: (i, 0, 0)),
        ] + sspecs + wspecs,
        out_specs=pl.BlockSpec((1, qb, outF), lambda i, j: (i, j, 0)),
        out_shape=jax.ShapeDtypeStruct((Bc, nq, outF), jnp.float32),
        compiler_params=_cparams(2),
    )(q_xyz, cand_xyz_t, table, *sargs, *wargs)


def _mlp(x, ws, relu_last=True):
    Bc, rows, _ = x.shape
    outF = ws[-1][0].shape[1]
    wargs = []
    wspecs = []
    for (W, b) in ws:
        wargs += [W, b.reshape(1, -1)]
        wspecs += [
            pl.BlockSpec(W.shape, lambda i: (0, 0)),
            pl.BlockSpec((1, b.shape[0]), lambda i: (0, 0)),
        ]

    def kern(x_ref, *rest):
        out_ref = rest[-1]
        h = x_ref[0]
        wl = list(zip(rest[0:-1:2], rest[1:-1:2]))
        for li, (W, b) in enumerate(wl):
            h = jnp.dot(h, W[...], preferred_element_type=jnp.float32) + b[...]
            if relu_last or li < len(wl) - 1:
                h = jnp.maximum(h, 0.0)
        out_ref[0] = h

    return pl.pallas_call(
        kern,
        grid=(Bc,),
        in_specs=[pl.BlockSpec((1, rows, x.shape[2]), lambda i: (i, 0, 0))]
        + wspecs,
        out_specs=pl.BlockSpec((1, rows, outF), lambda i: (i, 0, 0)),
        out_shape=jax.ShapeDtypeStruct((Bc, rows, outF), jnp.float32),
        compiler_params=_cparams(1),
    )(x, *wargs)


def _fp_cls(fine_xyz, coarse_xyz_t, coarse_feat, fine_feat, fp_ws, cls_ws):
    Bc, N, _ = fine_xyz.shape
    M, F = coarse_feat.shape[1], coarse_feat.shape[2]
    S = fine_feat.shape[2]
    qb = 2048
    nblk = N // qb
    layers = list(fp_ws) + list(cls_ws)
    n_relu = len(layers) - 1
    wargs = []
    wspecs = []
    for (W, b) in layers:
        wargs += [W, b.reshape(1, -1)]
        wspecs += [
            pl.BlockSpec(W.shape, lambda i, j: (0, 0)),
            pl.BlockSpec((1, b.shape[0]), lambda i, j: (0, 0)),
        ]

    def kern(fx_ref, cxt_ref, cf_ref, ff_ref, *rest):
        out_ref = rest[-1]
        wl = list(zip(rest[0:-1:2], rest[1:-1:2]))
        qx = fx_ref[0]
        dx = qx[:, 0:1] - cxt_ref[0, 0:1, :]
        dy = qx[:, 1:2] - cxt_ref[0, 1:2, :]
        dz = qx[:, 2:3] - cxt_ref[0, 2:3, :]
        d = dx * dx + dy * dy + dz * dz
        lane = _iota2((qb, M), 1)
        Wm = jnp.zeros((qb, M), jnp.float32)
        wsum = jnp.zeros((qb, 1), jnp.float32)
        for k in range(3):
            mval = jnp.min(d, axis=1, keepdims=True)
            cur = jnp.min(jnp.where(d == mval, lane, M), axis=1,
                          keepdims=True)
            d = jnp.where(lane == cur, _BIG, d)
            wk = 1.0 / (mval + 1e-10)
            Wm = Wm + wk * (lane == cur).astype(jnp.float32)
            wsum = wsum + wk
        Wm = Wm / wsum
        interp = jnp.dot(Wm, cf_ref[0], preferred_element_type=jnp.float32)
        h = jnp.concatenate([interp, ff_ref[0]], axis=1)
        for li, (W, b) in enumerate(wl):
            h = jnp.dot(h, W[...], preferred_element_type=jnp.float32) + b[...]
            if li < n_relu:
                h = jnp.maximum(h, 0.0)
        out_ref[0] = h

    return pl.pallas_call(
        kern,
        grid=(Bc, nblk),
        in_specs=[
            pl.BlockSpec((1, qb, 3), lambda i, j: (i, j, 0)),
            pl.BlockSpec((1, 3, M), lambda i, j: (i, 0, 0)),
            pl.BlockSpec((1, M, F), lambda i, j: (i, 0, 0)),
            pl.BlockSpec((1, qb, S), lambda i, j: (i, j, 0)),
        ] + wspecs,
        out_specs=pl.BlockSpec((1, qb, 3), lambda i, j: (i, j, 0)),
        out_shape=jax.ShapeDtypeStruct((Bc, N, 3), jnp.float32),
        compiler_params=_cparams(2),
    )(fine_xyz, coarse_xyz_t, coarse_feat, fine_feat, *wargs)


def _ball_select(q_xyz, cand_xyz_t, ns, r2, qb):
    Bc, nq, _ = q_xyz.shape
    nc = cand_xyz_t.shape[2]
    nblk = nq // qb

    def kern(q_ref, cxt_ref, out_ref):
        b = pl.program_id(0)
        qx = q_ref[0]
        dx = qx[:, 0:1] - cxt_ref[0, 0:1, :]
        dy = qx[:, 1:2] - cxt_ref[0, 1:2, :]
        dz = qx[:, 2:3] - cxt_ref[0, 2:3, :]
        d = dx * dx + dy * dy + dz * dz
        lane = _iota2((qb, nc), 1)
        mi = jnp.where(d <= r2, lane, nc)
        first = None
        for k in range(ns):
            cur = jnp.min(mi, axis=1, keepdims=True)
            mi = jnp.where(mi == cur, nc, mi)
            if k == 0:
                first = jnp.where(cur == nc, 0, cur)
                idx = first
            else:
                idx = jnp.where(cur == nc, first, cur)
            out_ref[0, 0, :, k : k + 1] = idx + b * nc

    return pl.pallas_call(
        kern,
        grid=(Bc, nblk),
        in_specs=[
            pl.BlockSpec((1, qb, 3), lambda i, j: (i, j, 0)),
            pl.BlockSpec((1, 3, nc), lambda i, j: (i, 0, 0)),
        ],
        out_specs=pl.BlockSpec((1, 1, qb, ns), lambda i, j: (i, j, 0, 0)),
        out_shape=jax.ShapeDtypeStruct((Bc, nblk, qb, ns), jnp.int32),
    )(q_xyz, cand_xyz_t)


def _sc_gather(table, idx):
    V, D = table.shape
    nrow = idx.shape[0]
    NW = 32
    b_per_w = nrow // NW
    nch = b_per_w // 128
    WAVE = 4
    idx2 = idx.reshape(NW * nch, 128)
    mesh = plsc.VectorSubcoreMesh(core_axis_name="c", subcore_axis_name="s")

    @functools.partial(
        pl.kernel,
        mesh=mesh,
        out_type=jax.ShapeDtypeStruct((nrow, D), jnp.float32),
        scratch_types=[
            pltpu.VMEM((nch, 128), jnp.int32),
            pltpu.VMEM((WAVE * 128, D), jnp.float32),
            pltpu.SemaphoreType.DMA,
        ],
    )
    def k(tab_hbm, idx_hbm, out_hbm, idx_v, rows_v, sem):
        wid = jax.lax.axis_index("s") * 2 + jax.lax.axis_index("c")
        pltpu.sync_copy(idx_hbm.at[pl.ds(wid * nch, nch)], idx_v)
        for w in range(nch // WAVE):
            cps = [
                pltpu.async_copy(
                    tab_hbm.at[idx_v.at[w * WAVE + j]],
                    rows_v.at[pl.ds(j * 128, 128)],
                    sem,
                )
                for j in range(WAVE)
            ]
            for cp in cps:
                cp.wait()
            pltpu.sync_copy(
                rows_v,
                out_hbm.at[pl.ds(wid * b_per_w + w * WAVE * 128,
                                 WAVE * 128)])

    return k(table, idx2)


def _post_group_mlp(rows, q_xyz, ws, ns, fw):
    Bc, nblk, qbns, Dp = rows.shape
    qb = qbns // ns
    nq = nblk * qb
    outF = ws[-1][0].shape[1]
    wargs = []
    wspecs = []
    for (W, b) in ws:
        wargs += [W, b.reshape(1, -1)]
        wspecs += [
            pl.BlockSpec(W.shape, lambda i, j: (0, 0)),
            pl.BlockSpec((1, b.shape[0]), lambda i, j: (0, 0)),
        ]

    def kern(r_ref, q_ref, *rest):
        out_ref = rest[-1]
        rows_b = r_ref[0, 0]
        qx = q_ref[0]
        qxrep = jnp.broadcast_to(qx[:, None, :], (qb, ns, 3)).reshape(
            qb * ns, 3)
        h = jnp.concatenate(
            [rows_b[:, 0:3] - qxrep, rows_b[:, 3:fw]], axis=1)
        for (W, b) in zip(rest[0:-1:2], rest[1:-1:2]):
            h = jnp.maximum(
                jnp.dot(h, W[...], preferred_element_type=jnp.float32)
                + b[...], 0.0)
        out_ref[0] = jnp.max(h.reshape(qb, ns, outF), axis=1)

    return pl.pallas_call(
        kern,
        grid=(Bc, nblk),
        in_specs=[
            pl.BlockSpec((1, 1, qbns, Dp), lambda i, j: (i, j, 0, 0)),
            pl.BlockSpec((1, qb, 3), lambda i, j: (i, j, 0)),
        ] + wspecs,
        out_specs=pl.BlockSpec((1, qb, outF), lambda i, j: (i, j, 0)),
        out_shape=jax.ShapeDtypeStruct((Bc, nq, outF), jnp.float32),
    )(rows, q_xyz, *wargs)


def kernel(points1, points2, features1, features2, params):
    P = params
    f1t = features1.transpose(0, 2, 1)
    f2t = features2.transpose(0, 2, 1)
    pts = jnp.concatenate([points1, points2], axis=0)
    ft = jnp.concatenate([f1t, f2t], axis=0)

    nx_t = _fps(pts, 256)
    nx = nx_t.transpose(0, 2, 1)
    B2, N = pts.shape[0], pts.shape[1]
    qb1, ns1 = 128, 16
    table1 = jnp.concatenate(
        [pts, ft, jnp.zeros((B2, N, 61), jnp.float32)], axis=-1)
    idx1 = _ball_select(nx, pts.transpose(0, 2, 1), ns=ns1, r2=1.0, qb=qb1)
    rows1 = _sc_gather(table1.reshape(B2 * N, 128), idx1.reshape(-1))
    rows1 = rows1.reshape(B2, 256 // qb1, qb1 * ns1, 128)
    f_sc1 = _post_group_mlp(rows1, nx, P['sc1'], ns=ns1, fw=67)
    B = points1.shape[0]
    p12, p22 = nx[:B], nx[B:]
    p12t, p22t = nx_t[:B], nx_t[B:]
    f12, f22 = f_sc1[:B], f_sc1[B:]

    table_fe = jnp.concatenate([p22, f22], axis=-1)
    emb = _group(p12, p22t, table_fe, P['fe'],
                 ns=64, r2=None, qb=128, self_feat=f12)

    p13t = _fps(p12, 64)
    p13 = p13t.transpose(0, 2, 1)
    table2 = jnp.concatenate([p12, emb], axis=-1)
    f13 = _group(p13, p12t, table2, P['sc2'], ns=8, r2=4.0, qb=64)

    p14t = _fps(p13, 16)
    p14 = p14t.transpose(0, 2, 1)
    table3 = jnp.concatenate([p13, f13], axis=-1)
    f14 = _group(p14, p13t, table3, P['sc3'], ns=8, r2=16.0, qb=16)

    t_up1 = jnp.concatenate([p14, f14], axis=-1)
    g1 = _group(p13, p14t, t_up1, [], ns=8, r2=None, qb=64)
    nf13 = _mlp(jnp.concatenate([g1, f13], axis=-1), P['up1_mlp2'])

    t_up2 = jnp.concatenate([p13, nf13], axis=-1)
    g2 = _group(p12, p13t, t_up2, P['up2_mlp1'], ns=8, r2=None, qb=256)
    skip_t = jnp.concatenate([f12, emb], axis=-1)
    nf12 = _mlp(jnp.concatenate([g2, skip_t], axis=-1), P['up2_mlp2'])

    out = _fp_cls(points1, p12t, nf12, f1t, P['fp'], P['cls'])
    return out.transpose(0, 2, 1)


# speedup vs baseline: 1.4885x; 1.0282x over previous
import functools

import jax
import jax.numpy as jnp
from jax.experimental import pallas as pl
from jax.experimental.pallas import tpu as pltpu
from jax.experimental.pallas import tpu_sc as plsc

_BIG = 1e10


def _cparams(n):
    return pltpu.CompilerParams(dimension_semantics=("parallel",) * n)


def _iota2(shape, dim):
    return jax.lax.broadcasted_iota(jnp.int32, shape, dim)


def _fps(xyz, npoint):
    Bc, N, _ = xyz.shape
    C = 128 if N >= 128 else N
    R = N // C
    planes = xyz.transpose(0, 2, 1).reshape(Bc, 3, R, C)

    def kern(p_ref, out_ref):
        P = p_ref[...]
        flat = (_iota2((Bc, R, C), 1) * C + _iota2((Bc, R, C), 2))
        lane = _iota2((Bc, 3, npoint), 2)

        def red(x, op):
            return op(op(x, axis=3, keepdims=True), axis=2, keepdims=True)

        def step(t, carry):
            dists, far, CO = carry
            sel = (flat == far)[:, None, :, :]
            cent = red(jnp.where(sel, P, 0.0), jnp.sum)
            CO = jnp.where(lane == t, cent[:, :, :, 0], CO)
            dd = P - cent
            dd = dd * dd
            d = dd[:, 0] + dd[:, 1] + dd[:, 2]
            dists = jnp.minimum(dists, d)
            m = jnp.max(jnp.max(dists, axis=2, keepdims=True), axis=1,
                        keepdims=True)
            far = jnp.min(jnp.min(jnp.where(dists == m, flat, N), axis=2,
                                  keepdims=True), axis=1, keepdims=True)
            return dists, far, CO

        init = (
            jnp.full((Bc, R, C), _BIG, jnp.float32),
            jnp.zeros((Bc, 1, 1), jnp.int32),
            jnp.zeros((Bc, 3, npoint), jnp.float32),
        )
        _, _, CO = jax.lax.fori_loop(0, npoint, step, init)
        out_ref[...] = CO

    return pl.pallas_call(
        kern,
        grid=(1,),
        in_specs=[pl.BlockSpec((Bc, 3, R, C), lambda i: (0, 0, 0, 0))],
        out_specs=pl.BlockSpec((Bc, 3, npoint), lambda i: (0, 0, 0)),
        out_shape=jax.ShapeDtypeStruct((Bc, 3, npoint), jnp.float32),
    )(planes)


def _group(q_xyz, cand_xyz_t, table, ws, ns, r2, qb, self_feat=None):
    Bc, nq, _ = q_xyz.shape
    nc, Ft = table.shape[1], table.shape[2]
    nblk = nq // qb
    S = 0 if self_feat is None else self_feat.shape[2]
    gw = 3 + S + (Ft - 3)
    outF = ws[-1][0].shape[1] if ws else gw

    wargs = []
    wspecs = []
    for (W, b) in ws:
        wargs += [W, b.reshape(1, -1)]
        wspecs += [
            pl.BlockSpec(W.shape, lambda i, j: (0, 0)),
            pl.BlockSpec((1, b.shape[0]), lambda i, j: (0, 0)),
        ]
    sargs = [] if self_feat is None else [self_feat]
    sspecs = [] if self_feat is None else [
        pl.BlockSpec((1, qb, S), lambda i, j: (i, j, 0))
    ]

    def kern(q_ref, cxt_ref, tab_ref, *rest):
        out_ref = rest[-1]
        rest = rest[:-1]
        self_blk = None
        if self_feat is not None:
            self_blk = rest[0][0]
            rest = rest[1:]
        qx = q_ref[0]
        dx = qx[:, 0:1] - cxt_ref[0, 0:1, :]
        dy = qx[:, 1:2] - cxt_ref[0, 1:2, :]
        dz = qx[:, 2:3] - cxt_ref[0, 2:3, :]
        d = dx * dx + dy * dy + dz * dz
        lane = _iota2((qb, nc), 1)

        ohs = []
        if r2 is not None:
            mi = jnp.where(d <= r2, lane, nc)
            first = None
            for k in range(ns):
                cur = jnp.min(mi, axis=1, keepdims=True)
                mi = jnp.where(mi == cur, nc, mi)
                if k == 0:
                    first = jnp.where(cur == nc, 0, cur)
                    idx = first
                else:
                    idx = jnp.where(cur == nc, first, cur)
                ohs.append((lane == idx).astype(jnp.float32))
        else:
            n3 = _iota2((qb, 1, nc), 2)
            R = jnp.zeros((qb, nc), jnp.float32)
            CH = 128 if nc > 128 else nc
            for m0 in range(0, nc, CH):
                dm = d[:, m0 : m0 + CH]
                m3 = _iota2((qb, CH, 1), 1) + m0
                lt = dm[:, :, None] < d[:, None, :]
                eq = dm[:, :, None] == d[:, None, :]
                cmp = lt | (eq & (m3 < n3))
                R = R + jnp.sum(cmp.astype(jnp.float32), axis=1)
            for k in range(ns):
                ohs.append((R == float(k)).astype(jnp.float32))

        tab = tab_ref[0]
        if self_blk is not None:
            W1, b1 = rest[0], rest[1]
            Wac = jnp.concatenate([W1[0:3, :], W1[3 + S :, :]], axis=0)
            A = jnp.dot(tab, Wac, preferred_element_type=jnp.float32)
            Bq = (jnp.dot(self_blk, W1[3 : 3 + S, :],
                          preferred_element_type=jnp.float32)
                  - jnp.dot(qx, W1[0:3, :],
                            preferred_element_type=jnp.float32) + b1[...])
            oh = jnp.concatenate(ohs, axis=0)
            h = jnp.maximum(
                jnp.dot(oh, A, preferred_element_type=jnp.float32)
                + jnp.concatenate([Bq] * ns, axis=0), 0.0)
            for (W, b) in zip(rest[2::2], rest[3::2]):
                h = jnp.maximum(
                    jnp.dot(h, W[...], preferred_element_type=jnp.float32)
                    + b[...], 0.0)
        else:
            if nc <= 512:
                oh = jnp.concatenate(ohs, axis=0)
                rows = jnp.dot(oh, tab, preferred_element_type=jnp.float32)
            else:
                rows = jnp.concatenate(
                    [jnp.dot(o, tab, preferred_element_type=jnp.float32)
                     for o in ohs], axis=0)

            qxt = jnp.concatenate([qx] * ns, axis=0)
            parts = [rows[:, 0:3] - qxt, rows[:, 3:]]
            h = jnp.concatenate(parts, axis=1)
            for (W, b) in zip(rest[0::2], rest[1::2]):
                h = jnp.maximum(
                    jnp.dot(h, W[...], preferred_element_type=jnp.float32)
                    + b[...], 0.0)
        h = h.reshape(ns, qb, outF)
        out_ref[0] = jnp.max(h, axis=0)

    return pl.pallas_call(
        kern,
        grid=(Bc, nblk),
        in_specs=[
            pl.BlockSpec((1, qb, 3), lambda i, j: (i, j, 0)),
            pl.BlockSpec((1, 3, nc), lambda i, j: (i, 0, 0)),
            pl.BlockSpec((1, nc, Ft), lambda i, j: (i, 0, 0)),
        ] + sspecs + wspecs,
        out_specs=pl.BlockSpec((1, qb, outF), lambda i, j: (i, j, 0)),
        out_shape=jax.ShapeDtypeStruct((Bc, nq, outF), jnp.float32),
        compiler_params=_cparams(2),
    )(q_xyz, cand_xyz_t, table, *sargs, *wargs)


def _mlp(x, ws, relu_last=True):
    Bc, rows, _ = x.shape
    outF = ws[-1][0].shape[1]
    wargs = []
    wspecs = []
    for (W, b) in ws:
        wargs += [W, b.reshape(1, -1)]
        wspecs += [
            pl.BlockSpec(W.shape, lambda i: (0, 0)),
            pl.BlockSpec((1, b.shape[0]), lambda i: (0, 0)),
        ]

    def kern(x_ref, *rest):
        out_ref = rest[-1]
        h = x_ref[0]
        wl = list(zip(rest[0:-1:2], rest[1:-1:2]))
        for li, (W, b) in enumerate(wl):
            h = jnp.dot(h, W[...], preferred_element_type=jnp.float32) + b[...]
            if relu_last or li < len(wl) - 1:
                h = jnp.maximum(h, 0.0)
        out_ref[0] = h

    return pl.pallas_call(
        kern,
        grid=(Bc,),
        in_specs=[pl.BlockSpec((1, rows, x.shape[2]), lambda i: (i, 0, 0))]
        + wspecs,
        out_specs=pl.BlockSpec((1, rows, outF), lambda i: (i, 0, 0)),
        out_shape=jax.ShapeDtypeStruct((Bc, rows, outF), jnp.float32),
        compiler_params=_cparams(1),
    )(x, *wargs)


def _fp_cls(fine_xyz, coarse_xyz_t, coarse_feat, fine_feat, fp_ws, cls_ws):
    Bc, N, _ = fine_xyz.shape
    M, F = coarse_feat.shape[1], coarse_feat.shape[2]
    S = fine_feat.shape[2]
    qb = 2048
    nblk = N // qb
    layers = list(fp_ws) + list(cls_ws)
    n_relu = len(layers) - 1
    wargs = []
    wspecs = []
    for (W, b) in layers:
        wargs += [W, b.reshape(1, -1)]
        wspecs += [
            pl.BlockSpec(W.shape, lambda i, j: (0, 0)),
            pl.BlockSpec((1, b.shape[0]), lambda i, j: (0, 0)),
        ]

    def kern(fx_ref, cxt_ref, cf_ref, ff_ref, *rest):
        out_ref = rest[-1]
        wl = list(zip(rest[0:-1:2], rest[1:-1:2]))
        qx = fx_ref[0]
        dx = qx[:, 0:1] - cxt_ref[0, 0:1, :]
        dy = qx[:, 1:2] - cxt_ref[0, 1:2, :]
        dz = qx[:, 2:3] - cxt_ref[0, 2:3, :]
        d = dx * dx + dy * dy + dz * dz
        lane = _iota2((qb, M), 1)
        Wm = jnp.zeros((qb, M), jnp.float32)
        wsum = jnp.zeros((qb, 1), jnp.float32)
        for k in range(3):
            mval = jnp.min(d, axis=1, keepdims=True)
            cur = jnp.min(jnp.where(d == mval, lane, M), axis=1,
                          keepdims=True)
            d = jnp.where(lane == cur, _BIG, d)
            wk = 1.0 / (mval + 1e-10)
            Wm = Wm + wk * (lane == cur).astype(jnp.float32)
            wsum = wsum + wk
        Wm = Wm / wsum
        interp = jnp.dot(Wm, cf_ref[0], preferred_element_type=jnp.float32)
        h = jnp.concatenate([interp, ff_ref[0]], axis=1)
        for li, (W, b) in enumerate(wl):
            h = jnp.dot(h, W[...], preferred_element_type=jnp.float32) + b[...]
            if li < n_relu:
                h = jnp.maximum(h, 0.0)
        out_ref[0] = h

    return pl.pallas_call(
        kern,
        grid=(Bc, nblk),
        in_specs=[
            pl.BlockSpec((1, qb, 3), lambda i, j: (i, j, 0)),
            pl.BlockSpec((1, 3, M), lambda i, j: (i, 0, 0)),
            pl.BlockSpec((1, M, F), lambda i, j: (i, 0, 0)),
            pl.BlockSpec((1, qb, S), lambda i, j: (i, j, 0)),
        ] + wspecs,
        out_specs=pl.BlockSpec((1, qb, 3), lambda i, j: (i, j, 0)),
        out_shape=jax.ShapeDtypeStruct((Bc, N, 3), jnp.float32),
        compiler_params=_cparams(2),
    )(fine_xyz, coarse_xyz_t, coarse_feat, fine_feat, *wargs)


def _ball_select(q_xyz, cand_xyz_t, ns, r2, qb):
    Bc, nq, _ = q_xyz.shape
    nc = cand_xyz_t.shape[2]
    nblk = nq // qb

    def kern(q_ref, cxt_ref, out_ref):
        b = pl.program_id(0)
        qx = q_ref[0]
        dx = qx[:, 0:1] - cxt_ref[0, 0:1, :]
        dy = qx[:, 1:2] - cxt_ref[0, 1:2, :]
        dz = qx[:, 2:3] - cxt_ref[0, 2:3, :]
        d = dx * dx + dy * dy + dz * dz
        lane = _iota2((qb, nc), 1)
        mi = jnp.where(d <= r2, lane, nc)
        first = None
        cur = None
        for k in range(ns):
            if k == 0:
                cur = jnp.min(mi, axis=1, keepdims=True)
                first = jnp.where(cur == nc, 0, cur)
                idx = first
            else:
                cur = jnp.min(jnp.where(mi > cur, mi, nc), axis=1,
                              keepdims=True)
                idx = jnp.where(cur == nc, first, cur)
            out_ref[0, 0, :, k : k + 1] = idx + b * nc

    return pl.pallas_call(
        kern,
        grid=(Bc, nblk),
        in_specs=[
            pl.BlockSpec((1, qb, 3), lambda i, j: (i, j, 0)),
            pl.BlockSpec((1, 3, nc), lambda i, j: (i, 0, 0)),
        ],
        out_specs=pl.BlockSpec((1, 1, qb, ns), lambda i, j: (i, j, 0, 0)),
        out_shape=jax.ShapeDtypeStruct((Bc, nblk, qb, ns), jnp.int32),
    )(q_xyz, cand_xyz_t)


def _sc_gather(table, idx):
    V, D = table.shape
    nrow = idx.shape[0]
    NW = 32
    b_per_w = nrow // NW
    nch = b_per_w // 128
    WAVE = 4
    idx2 = idx.reshape(NW * nch, 128)
    mesh = plsc.VectorSubcoreMesh(core_axis_name="c", subcore_axis_name="s")

    @functools.partial(
        pl.kernel,
        mesh=mesh,
        out_type=jax.ShapeDtypeStruct((nrow, D), jnp.float32),
        scratch_types=[
            pltpu.VMEM((nch, 128), jnp.int32),
            pltpu.VMEM((WAVE * 128, D), jnp.float32),
            pltpu.SemaphoreType.DMA,
        ],
    )
    def k(tab_hbm, idx_hbm, out_hbm, idx_v, rows_v, sem):
        wid = jax.lax.axis_index("s") * 2 + jax.lax.axis_index("c")
        pltpu.sync_copy(idx_hbm.at[pl.ds(wid * nch, nch)], idx_v)
        for w in range(nch // WAVE):
            cps = [
                pltpu.async_copy(
                    tab_hbm.at[idx_v.at[w * WAVE + j]],
                    rows_v.at[pl.ds(j * 128, 128)],
                    sem,
                )
                for j in range(WAVE)
            ]
            for cp in cps:
                cp.wait()
            pltpu.sync_copy(
                rows_v,
                out_hbm.at[pl.ds(wid * b_per_w + w * WAVE * 128,
                                 WAVE * 128)])

    return k(table, idx2)


def _post_group_mlp(rows, q_xyz, ws, ns, fw):
    Bc, nblk, qbns, Dp = rows.shape
    qb = qbns // ns
    nq = nblk * qb
    outF = ws[-1][0].shape[1]
    wargs = []
    wspecs = []
    for (W, b) in ws:
        wargs += [W, b.reshape(1, -1)]
        wspecs += [
            pl.BlockSpec(W.shape, lambda i, j: (0, 0)),
            pl.BlockSpec((1, b.shape[0]), lambda i, j: (0, 0)),
        ]

    def kern(r_ref, q_ref, *rest):
        out_ref = rest[-1]
        rows_b = r_ref[0, 0]
        qx = q_ref[0]
        qxrep = jnp.broadcast_to(qx[:, None, :], (qb, ns, 3)).reshape(
            qb * ns, 3)
        h = jnp.concatenate(
            [rows_b[:, 0:3] - qxrep, rows_b[:, 3:fw]], axis=1)
        for (W, b) in zip(rest[0:-1:2], rest[1:-1:2]):
            h = jnp.maximum(
                jnp.dot(h, W[...], preferred_element_type=jnp.float32)
                + b[...], 0.0)
        out_ref[0] = jnp.max(h.reshape(qb, ns, outF), axis=1)

    return pl.pallas_call(
        kern,
        grid=(Bc, nblk),
        in_specs=[
            pl.BlockSpec((1, 1, qbns, Dp), lambda i, j: (i, j, 0, 0)),
            pl.BlockSpec((1, qb, 3), lambda i, j: (i, j, 0)),
        ] + wspecs,
        out_specs=pl.BlockSpec((1, qb, outF), lambda i, j: (i, j, 0)),
        out_shape=jax.ShapeDtypeStruct((Bc, nq, outF), jnp.float32),
    )(rows, q_xyz, *wargs)


def kernel(points1, points2, features1, features2, params):
    P = params
    f1t = features1.transpose(0, 2, 1)
    f2t = features2.transpose(0, 2, 1)
    pts = jnp.concatenate([points1, points2], axis=0)
    ft = jnp.concatenate([f1t, f2t], axis=0)

    nx_t = _fps(pts, 256)
    nx = nx_t.transpose(0, 2, 1)
    B2, N = pts.shape[0], pts.shape[1]
    qb1, ns1 = 256, 16
    table1 = jnp.concatenate(
        [pts, ft, jnp.zeros((B2, N, 61), jnp.float32)], axis=-1)
    idx1 = _ball_select(nx, pts.transpose(0, 2, 1), ns=ns1, r2=1.0, qb=qb1)
    rows1 = _sc_gather(table1.reshape(B2 * N, 128), idx1.reshape(-1))
    rows1 = rows1.reshape(B2, 256 // qb1, qb1 * ns1, 128)
    f_sc1 = _post_group_mlp(rows1, nx, P['sc1'], ns=ns1, fw=67)
    B = points1.shape[0]
    p12, p22 = nx[:B], nx[B:]
    p12t, p22t = nx_t[:B], nx_t[B:]
    f12, f22 = f_sc1[:B], f_sc1[B:]

    table_fe = jnp.concatenate([p22, f22], axis=-1)
    emb = _group(p12, p22t, table_fe, P['fe'],
                 ns=64, r2=None, qb=128, self_feat=f12)

    p13t = _fps(p12, 64)
    p13 = p13t.transpose(0, 2, 1)
    table2 = jnp.concatenate([p12, emb], axis=-1)
    f13 = _group(p13, p12t, table2, P['sc2'], ns=8, r2=4.0, qb=64)

    p14t = _fps(p13, 16)
    p14 = p14t.transpose(0, 2, 1)
    table3 = jnp.concatenate([p13, f13], axis=-1)
    f14 = _group(p14, p13t, table3, P['sc3'], ns=8, r2=16.0, qb=16)

    t_up1 = jnp.concatenate([p14, f14], axis=-1)
    g1 = _group(p13, p14t, t_up1, [], ns=8, r2=None, qb=64)
    nf13 = _mlp(jnp.concatenate([g1, f13], axis=-1), P['up1_mlp2'])

    t_up2 = jnp.concatenate([p13, nf13], axis=-1)
    g2 = _group(p12, p13t, t_up2, P['up2_mlp1'], ns=8, r2=None, qb=256)
    skip_t = jnp.concatenate([f12, emb], axis=-1)
    nf12 = _mlp(jnp.concatenate([g2, skip_t], axis=-1), P['up2_mlp2'])

    out = _fp_cls(points1, p12t, nf12, f1t, P['fp'], P['cls'])
    return out.transpose(0, 2, 1)


# speedup vs baseline: 1.8238x; 1.2253x over previous
import functools

import jax
import jax.numpy as jnp
from jax.experimental import pallas as pl
from jax.experimental.pallas import tpu as pltpu
from jax.experimental.pallas import tpu_sc as plsc

_BIG = 1e10


def _cparams(n):
    return pltpu.CompilerParams(dimension_semantics=("parallel",) * n)


def _iota2(shape, dim):
    return jax.lax.broadcasted_iota(jnp.int32, shape, dim)


def _fps(xyz, npoint):
    Bc, N, _ = xyz.shape
    C = 128 if N >= 128 else N
    R = N // C
    planes = xyz.transpose(0, 2, 1).reshape(Bc, 3, R, C)

    def kern(p_ref, out_ref):
        P = p_ref[...]
        flat = (_iota2((Bc, R, C), 1) * C + _iota2((Bc, R, C), 2))
        lane = _iota2((Bc, 3, npoint), 2)

        def red(x, op):
            return op(op(x, axis=3, keepdims=True), axis=2, keepdims=True)

        def step(t, carry):
            dists, far, CO = carry
            sel = (flat == far)[:, None, :, :]
            cent = red(jnp.where(sel, P, 0.0), jnp.sum)
            CO = jnp.where(lane == t, cent[:, :, :, 0], CO)
            dd = P - cent
            dd = dd * dd
            d = dd[:, 0] + dd[:, 1] + dd[:, 2]
            dists = jnp.minimum(dists, d)
            m = jnp.max(jnp.max(dists, axis=2, keepdims=True), axis=1,
                        keepdims=True)
            far = jnp.min(jnp.min(jnp.where(dists == m, flat, N), axis=2,
                                  keepdims=True), axis=1, keepdims=True)
            return dists, far, CO

        init = (
            jnp.full((Bc, R, C), _BIG, jnp.float32),
            jnp.zeros((Bc, 1, 1), jnp.int32),
            jnp.zeros((Bc, 3, npoint), jnp.float32),
        )
        _, _, CO = jax.lax.fori_loop(0, npoint, step, init)
        out_ref[...] = CO

    return pl.pallas_call(
        kern,
        grid=(1,),
        in_specs=[pl.BlockSpec((Bc, 3, R, C), lambda i: (0, 0, 0, 0))],
        out_specs=pl.BlockSpec((Bc, 3, npoint), lambda i: (0, 0, 0)),
        out_shape=jax.ShapeDtypeStruct((Bc, 3, npoint), jnp.float32),
    )(planes)


def _group(q_xyz, cand_xyz_t, table, ws, ns, r2, qb, self_feat=None):
    Bc, nq, _ = q_xyz.shape
    nc, Ft = table.shape[1], table.shape[2]
    nblk = nq // qb
    S = 0 if self_feat is None else self_feat.shape[2]
    gw = 3 + S + (Ft - 3)
    outF = ws[-1][0].shape[1] if ws else gw

    wargs = []
    wspecs = []
    for (W, b) in ws:
        wargs += [W, b.reshape(1, -1)]
        wspecs += [
            pl.BlockSpec(W.shape, lambda i, j: (0, 0)),
            pl.BlockSpec((1, b.shape[0]), lambda i, j: (0, 0)),
        ]
    sargs = [] if self_feat is None else [self_feat]
    sspecs = [] if self_feat is None else [
        pl.BlockSpec((1, qb, S), lambda i, j: (i, j, 0))
    ]

    def kern(q_ref, cxt_ref, tab_ref, *rest):
        out_ref = rest[-1]
        rest = rest[:-1]
        self_blk = None
        if self_feat is not None:
            self_blk = rest[0][0]
            rest = rest[1:]
        qx = q_ref[0]
        dx = qx[:, 0:1] - cxt_ref[0, 0:1, :]
        dy = qx[:, 1:2] - cxt_ref[0, 1:2, :]
        dz = qx[:, 2:3] - cxt_ref[0, 2:3, :]
        d = dx * dx + dy * dy + dz * dz
        lane = _iota2((qb, nc), 1)

        ohs = []
        if r2 is not None:
            mi = jnp.where(d <= r2, lane, nc)
            first = None
            for k in range(ns):
                cur = jnp.min(mi, axis=1, keepdims=True)
                mi = jnp.where(mi == cur, nc, mi)
                if k == 0:
                    first = jnp.where(cur == nc, 0, cur)
                    idx = first
                else:
                    idx = jnp.where(cur == nc, first, cur)
                ohs.append((lane == idx).astype(jnp.float32))
        else:
            n3 = _iota2((qb, 1, nc), 2)
            R = jnp.zeros((qb, nc), jnp.float32)
            CH = 128 if nc > 128 else nc
            for m0 in range(0, nc, CH):
                dm = d[:, m0 : m0 + CH]
                m3 = _iota2((qb, CH, 1), 1) + m0
                lt = dm[:, :, None] < d[:, None, :]
                eq = dm[:, :, None] == d[:, None, :]
                cmp = lt | (eq & (m3 < n3))
                R = R + jnp.sum(cmp.astype(jnp.float32), axis=1)
            for k in range(ns):
                ohs.append((R == float(k)).astype(jnp.float32))

        tab = tab_ref[0]
        if self_blk is not None:
            W1, b1 = rest[0], rest[1]
            Wac = jnp.concatenate([W1[0:3, :], W1[3 + S :, :]], axis=0)
            A = jnp.dot(tab, Wac, preferred_element_type=jnp.float32)
            Bq = (jnp.dot(self_blk, W1[3 : 3 + S, :],
                          preferred_element_type=jnp.float32)
                  - jnp.dot(qx, W1[0:3, :],
                            preferred_element_type=jnp.float32) + b1[...])
            oh = jnp.concatenate(ohs, axis=0)
            h = jnp.maximum(
                jnp.dot(oh, A, preferred_element_type=jnp.float32)
                + jnp.concatenate([Bq] * ns, axis=0), 0.0)
            for (W, b) in zip(rest[2::2], rest[3::2]):
                h = jnp.maximum(
                    jnp.dot(h, W[...], preferred_element_type=jnp.float32)
                    + b[...], 0.0)
        else:
            if nc <= 512:
                oh = jnp.concatenate(ohs, axis=0)
                rows = jnp.dot(oh, tab, preferred_element_type=jnp.float32)
            else:
                rows = jnp.concatenate(
                    [jnp.dot(o, tab, preferred_element_type=jnp.float32)
                     for o in ohs], axis=0)

            qxt = jnp.concatenate([qx] * ns, axis=0)
            parts = [rows[:, 0:3] - qxt, rows[:, 3:]]
            h = jnp.concatenate(parts, axis=1)
            for (W, b) in zip(rest[0::2], rest[1::2]):
                h = jnp.maximum(
                    jnp.dot(h, W[...], preferred_element_type=jnp.float32)
                    + b[...], 0.0)
        h = h.reshape(ns, qb, outF)
        out_ref[0] = jnp.max(h, axis=0)

    return pl.pallas_call(
        kern,
        grid=(Bc, nblk),
        in_specs=[
            pl.BlockSpec((1, qb, 3), lambda i, j: (i, j, 0)),
            pl.BlockSpec((1, 3, nc), lambda i, j: (i, 0, 0)),
            pl.BlockSpec((1, nc, Ft), lambda i, j: (i, 0, 0)),
        ] + sspecs + wspecs,
        out_specs=pl.BlockSpec((1, qb, outF), lambda i, j: (i, j, 0)),
        out_shape=jax.ShapeDtypeStruct((Bc, nq, outF), jnp.float32),
        compiler_params=_cparams(2),
    )(q_xyz, cand_xyz_t, table, *sargs, *wargs)


def _mlp(x, ws, relu_last=True):
    Bc, rows, _ = x.shape
    outF = ws[-1][0].shape[1]
    wargs = []
    wspecs = []
    for (W, b) in ws:
        wargs += [W, b.reshape(1, -1)]
        wspecs += [
            pl.BlockSpec(W.shape, lambda i: (0, 0)),
            pl.BlockSpec((1, b.shape[0]), lambda i: (0, 0)),
        ]

    def kern(x_ref, *rest):
        out_ref = rest[-1]
        h = x_ref[0]
        wl = list(zip(rest[0:-1:2], rest[1:-1:2]))
        for li, (W, b) in enumerate(wl):
            h = jnp.dot(h, W[...], preferred_element_type=jnp.float32) + b[...]
            if relu_last or li < len(wl) - 1:
                h = jnp.maximum(h, 0.0)
        out_ref[0] = h

    return pl.pallas_call(
        kern,
        grid=(Bc,),
        in_specs=[pl.BlockSpec((1, rows, x.shape[2]), lambda i: (i, 0, 0))]
        + wspecs,
        out_specs=pl.BlockSpec((1, rows, outF), lambda i: (i, 0, 0)),
        out_shape=jax.ShapeDtypeStruct((Bc, rows, outF), jnp.float32),
        compiler_params=_cparams(1),
    )(x, *wargs)


def _fp_cls(fine_xyz, coarse_xyz_t, coarse_feat, fine_feat, fp_ws, cls_ws):
    Bc, N, _ = fine_xyz.shape
    M, F = coarse_feat.shape[1], coarse_feat.shape[2]
    S = fine_feat.shape[2]
    qb = 2048
    nblk = N // qb
    layers = list(fp_ws) + list(cls_ws)
    n_relu = len(layers) - 1
    wargs = []
    wspecs = []
    for (W, b) in layers:
        wargs += [W, b.reshape(1, -1)]
        wspecs += [
            pl.BlockSpec(W.shape, lambda i, j: (0, 0)),
            pl.BlockSpec((1, b.shape[0]), lambda i, j: (0, 0)),
        ]

    def kern(fx_ref, cxt_ref, cf_ref, ff_ref, *rest):
        out_ref = rest[-1]
        wl = list(zip(rest[0:-1:2], rest[1:-1:2]))
        qx = fx_ref[0]
        dx = qx[:, 0:1] - cxt_ref[0, 0:1, :]
        dy = qx[:, 1:2] - cxt_ref[0, 1:2, :]
        dz = qx[:, 2:3] - cxt_ref[0, 2:3, :]
        d = dx * dx + dy * dy + dz * dz
        lane = _iota2((qb, M), 1)
        Wm = jnp.zeros((qb, M), jnp.float32)
        wsum = jnp.zeros((qb, 1), jnp.float32)
        for k in range(3):
            mval = jnp.min(d, axis=1, keepdims=True)
            cur = jnp.min(jnp.where(d == mval, lane, M), axis=1,
                          keepdims=True)
            d = jnp.where(lane == cur, _BIG, d)
            wk = 1.0 / (mval + 1e-10)
            Wm = Wm + wk * (lane == cur).astype(jnp.float32)
            wsum = wsum + wk
        Wm = Wm / wsum
        interp = jnp.dot(Wm, cf_ref[0], preferred_element_type=jnp.float32)
        h = jnp.concatenate([interp, ff_ref[0]], axis=1)
        for li, (W, b) in enumerate(wl):
            h = jnp.dot(h, W[...], preferred_element_type=jnp.float32) + b[...]
            if li < n_relu:
                h = jnp.maximum(h, 0.0)
        out_ref[0] = h

    return pl.pallas_call(
        kern,
        grid=(Bc, nblk),
        in_specs=[
            pl.BlockSpec((1, qb, 3), lambda i, j: (i, j, 0)),
            pl.BlockSpec((1, 3, M), lambda i, j: (i, 0, 0)),
            pl.BlockSpec((1, M, F), lambda i, j: (i, 0, 0)),
            pl.BlockSpec((1, qb, S), lambda i, j: (i, j, 0)),
        ] + wspecs,
        out_specs=pl.BlockSpec((1, qb, 3), lambda i, j: (i, j, 0)),
        out_shape=jax.ShapeDtypeStruct((Bc, N, 3), jnp.float32),
        compiler_params=_cparams(2),
    )(fine_xyz, coarse_xyz_t, coarse_feat, fine_feat, *wargs)


def _ball_select(q_xyz, cand_xyz_t, ns, r2, qb):
    Bc, nq, _ = q_xyz.shape
    nc = cand_xyz_t.shape[2]
    nblk = nq // qb

    W = 1024 if nc >= 2048 else nc

    def kern(q_ref, cxt_ref, out_ref):
        b = pl.program_id(0)
        qx = q_ref[0]

        def masked_iota(lo, w):
            dx = qx[:, 0:1] - cxt_ref[0, 0:1, lo : lo + w]
            dy = qx[:, 1:2] - cxt_ref[0, 1:2, lo : lo + w]
            dz = qx[:, 2:3] - cxt_ref[0, 2:3, lo : lo + w]
            d = dx * dx + dy * dy + dz * dz
            lane = _iota2((qb, w), 1) + lo
            return jnp.where(d <= r2, lane, nc)

        def extract(mi):
            first = None
            cur = None
            for k in range(ns):
                if k == 0:
                    cur = jnp.min(mi, axis=1, keepdims=True)
                    first = jnp.where(cur == nc, 0, cur)
                    idx = first
                else:
                    cur = jnp.min(jnp.where(mi > cur, mi, nc), axis=1,
                                  keepdims=True)
                    idx = jnp.where(cur == nc, first, cur)
                out_ref[0, 0, :, k : k + 1] = idx + b * nc

        mi_w = masked_iota(0, W)
        if W == nc:
            extract(mi_w)
        else:
            cnt = jnp.sum((mi_w < nc).astype(jnp.float32), axis=1,
                          keepdims=True)
            mincnt = jnp.min(cnt, axis=0, keepdims=True)
            full = mincnt[0, 0] < float(ns)

            @pl.when(jnp.logical_not(full))
            def _():
                extract(mi_w)

            @pl.when(full)
            def _():
                extract(jnp.concatenate(
                    [mi_w, masked_iota(W, nc - W)], axis=1))

    return pl.pallas_call(
        kern,
        grid=(Bc, nblk),
        in_specs=[
            pl.BlockSpec((1, qb, 3), lambda i, j: (i, j, 0)),
            pl.BlockSpec((1, 3, nc), lambda i, j: (i, 0, 0)),
        ],
        out_specs=pl.BlockSpec((1, 1, qb, ns), lambda i, j: (i, j, 0, 0)),
        out_shape=jax.ShapeDtypeStruct((Bc, nblk, qb, ns), jnp.int32),
    )(q_xyz, cand_xyz_t)


def _sc_gather(table, idx):
    V, D = table.shape
    nrow = idx.shape[0]
    NW = 32
    b_per_w = nrow // NW
    nch = b_per_w // 128
    WAVE = 4
    idx2 = idx.reshape(NW * nch, 128)
    mesh = plsc.VectorSubcoreMesh(core_axis_name="c", subcore_axis_name="s")

    @functools.partial(
        pl.kernel,
        mesh=mesh,
        out_type=jax.ShapeDtypeStruct((nrow, D), jnp.float32),
        scratch_types=[
            pltpu.VMEM((nch, 128), jnp.int32),
            pltpu.VMEM((WAVE * 128, D), jnp.float32),
            pltpu.SemaphoreType.DMA,
        ],
    )
    def k(tab_hbm, idx_hbm, out_hbm, idx_v, rows_v, sem):
        wid = jax.lax.axis_index("s") * 2 + jax.lax.axis_index("c")
        pltpu.sync_copy(idx_hbm.at[pl.ds(wid * nch, nch)], idx_v)
        for w in range(nch // WAVE):
            cps = [
                pltpu.async_copy(
                    tab_hbm.at[idx_v.at[w * WAVE + j]],
                    rows_v.at[pl.ds(j * 128, 128)],
                    sem,
                )
                for j in range(WAVE)
            ]
            for cp in cps:
                cp.wait()
            pltpu.sync_copy(
                rows_v,
                out_hbm.at[pl.ds(wid * b_per_w + w * WAVE * 128,
                                 WAVE * 128)])

    return k(table, idx2)


def _post_group_mlp(rows, q_xyz, ws, ns, fw):
    Bc, nblk, qbns, Dp = rows.shape
    qb = qbns // ns
    nq = nblk * qb
    outF = ws[-1][0].shape[1]
    wargs = []
    wspecs = []
    for (W, b) in ws:
        wargs += [W, b.reshape(1, -1)]
        wspecs += [
            pl.BlockSpec(W.shape, lambda i, j: (0, 0)),
            pl.BlockSpec((1, b.shape[0]), lambda i, j: (0, 0)),
        ]

    def kern(r_ref, q_ref, *rest):
        out_ref = rest[-1]
        rows_b = r_ref[0, 0]
        qx = q_ref[0]
        qxrep = jnp.broadcast_to(qx[:, None, :], (qb, ns, 3)).reshape(
            qb * ns, 3)
        h = jnp.concatenate(
            [rows_b[:, 0:3] - qxrep, rows_b[:, 3:fw]], axis=1)
        for (W, b) in zip(rest[0:-1:2], rest[1:-1:2]):
            h = jnp.maximum(
                jnp.dot(h, W[...], preferred_element_type=jnp.float32)
                + b[...], 0.0)
        out_ref[0] = jnp.max(h.reshape(qb, ns, outF), axis=1)

    return pl.pallas_call(
        kern,
        grid=(Bc, nblk),
        in_specs=[
            pl.BlockSpec((1, 1, qbns, Dp), lambda i, j: (i, j, 0, 0)),
            pl.BlockSpec((1, qb, 3), lambda i, j: (i, j, 0)),
        ] + wspecs,
        out_specs=pl.BlockSpec((1, qb, outF), lambda i, j: (i, j, 0)),
        out_shape=jax.ShapeDtypeStruct((Bc, nq, outF), jnp.float32),
    )(rows, q_xyz, *wargs)


def kernel(points1, points2, features1, features2, params):
    P = params
    f1t = features1.transpose(0, 2, 1)
    f2t = features2.transpose(0, 2, 1)
    pts = jnp.concatenate([points1, points2], axis=0)
    ft = jnp.concatenate([f1t, f2t], axis=0)

    nx_t = _fps(pts, 256)
    nx = nx_t.transpose(0, 2, 1)
    B2, N = pts.shape[0], pts.shape[1]
    qb1, ns1 = 256, 16
    table1 = jnp.concatenate(
        [pts, ft, jnp.zeros((B2, N, 61), jnp.float32)], axis=-1)
    idx1 = _ball_select(nx, pts.transpose(0, 2, 1), ns=ns1, r2=1.0, qb=qb1)
    rows1 = _sc_gather(table1.reshape(B2 * N, 128), idx1.reshape(-1))
    rows1 = rows1.reshape(B2, 256 // qb1, qb1 * ns1, 128)
    f_sc1 = _post_group_mlp(rows1, nx, P['sc1'], ns=ns1, fw=67)
    B = points1.shape[0]
    p12, p22 = nx[:B], nx[B:]
    p12t, p22t = nx_t[:B], nx_t[B:]
    f12, f22 = f_sc1[:B], f_sc1[B:]

    table_fe = jnp.concatenate([p22, f22], axis=-1)
    emb = _group(p12, p22t, table_fe, P['fe'],
                 ns=64, r2=None, qb=128, self_feat=f12)

    p13t = _fps(p12, 64)
    p13 = p13t.transpose(0, 2, 1)
    table2 = jnp.concatenate([p12, emb], axis=-1)
    f13 = _group(p13, p12t, table2, P['sc2'], ns=8, r2=4.0, qb=64)

    p14t = _fps(p13, 16)
    p14 = p14t.transpose(0, 2, 1)
    table3 = jnp.concatenate([p13, f13], axis=-1)
    f14 = _group(p14, p13t, table3, P['sc3'], ns=8, r2=16.0, qb=16)

    t_up1 = jnp.concatenate([p14, f14], axis=-1)
    g1 = _group(p13, p14t, t_up1, [], ns=8, r2=None, qb=64)
    nf13 = _mlp(jnp.concatenate([g1, f13], axis=-1), P['up1_mlp2'])

    t_up2 = jnp.concatenate([p13, nf13], axis=-1)
    g2 = _group(p12, p13t, t_up2, P['up2_mlp1'], ns=8, r2=None, qb=256)
    skip_t = jnp.concatenate([f12, emb], axis=-1)
    nf12 = _mlp(jnp.concatenate([g2, skip_t], axis=-1), P['up2_mlp2'])

    out = _fp_cls(points1, p12t, nf12, f1t, P['fp'], P['cls'])
    return out.transpose(0, 2, 1)
